# Initial kernel scaffold; baseline (speedup 1.0000x reference)
import jax
import jax.numpy as jnp
from jax.experimental import pallas as pl


def kernel(x, edge_index, edge_attr, batch, params):
    raise NotImplementedError("write your pallas kernel here")


# speedup vs baseline: 1.6545x; 1.6545x over previous
import functools
import math

import jax
import jax.numpy as jnp
from jax import lax
from jax.experimental import pallas as pl
from jax.experimental.pallas import tpu as pltpu
from jax.experimental.pallas import tpu_sc as plsc

H = 128
NG = 64
N = 10000
NE = 320000
NB = 128
NPAD = 10240
NBLK = NPAD // NB
EB = 512
EBLK = NE // EB
NW = 32
PERW = NE // NW
CH = 80
NCH = PERW // CH
ROWS_PER_TILE = NPAD // 16

_BNS = 1.0 / math.sqrt(1.0 + 1e-5)


def _f32(x):
    return x.astype(jnp.float32)


def _dot(a, b):
    return jax.lax.dot_general(a, b, (((1,), (0,)), ((), ())),
                               precision=jax.lax.Precision.HIGHEST,
                               preferred_element_type=jnp.float32)


def _dotd(a, b):
    return jax.lax.dot_general(a, b, (((1,), (0,)), ((), ())),
                               preferred_element_type=jnp.float32)


def _enc_body(xb, brow, iew, ieb, ieg, iebe, x0, cnt, acc):
    i = pl.program_id(0)

    @pl.when(i == 0)
    def _():
        acc[...] = jnp.zeros_like(acc)

    oh = (lax.broadcasted_iota(jnp.int32, (NG, NB), 0) == brow[...]
          ).astype(jnp.float32)
    acc[...] += jnp.sum(oh, axis=1, keepdims=True)
    y = _dotd(xb[...], iew[...]) + ieb[...]
    x0[...] = jnp.maximum(y * (ieg[...] * _BNS) + iebe[...], 0.0)

    @pl.when(i == NBLK - 1)
    def _():
        cnt[...] = jnp.maximum(acc[...], 1.0)


def _encoder(xp, brow, p):
    return pl.pallas_call(
        _enc_body,
        grid=(NBLK,),
        in_specs=[
            pl.BlockSpec((NB, H), lambda i: (i, 0)),
            pl.BlockSpec((1, NB), lambda i: (0, i)),
            pl.BlockSpec((H, H), lambda i: (0, 0)),
            pl.BlockSpec((1, H), lambda i: (0, 0)),
            pl.BlockSpec((1, H), lambda i: (0, 0)),
            pl.BlockSpec((1, H), lambda i: (0, 0)),
        ],
        out_specs=[
            pl.BlockSpec((NB, H), lambda i: (i, 0)),
            pl.BlockSpec((NG, 1), lambda i: (0, 0)),
        ],
        out_shape=[
            jax.ShapeDtypeStruct((NPAD, H), jnp.float32),
            jax.ShapeDtypeStruct((NG, 1), jnp.float32),
        ],
        scratch_shapes=[pltpu.VMEM((NG, 1), jnp.float32)],
    )(xp, brow, p['ie_w'], p['ie_b'].reshape(1, H),
      p['ie_g'].reshape(1, H), p['ie_be'].reshape(1, H))


def _vn_body(xb, brow, cnt, vemb, w1, b1, g, be, w2, b2, a1wb, a1b,
             vnu_out, vrow_out, acc):
    i = pl.program_id(0)

    @pl.when(i == 0)
    def _():
        acc[...] = jnp.zeros_like(acc)

    oh = (lax.broadcasted_iota(jnp.int32, (NG, NB), 0) == brow[...]
          ).astype(jnp.float32)
    acc[...] += _dot(oh, xb[...])

    @pl.when(i == NBLK - 1)
    def _():
        vn_in = acc[...] / cnt[...]
        z = vemb[...] + vn_in
        h = jnp.maximum((_dotd(z, w1[...]) + b1[...]) * (g[...] * _BNS)
                        + be[...], 0.0)
        vnu = _dotd(h, w2[...]) + b2[...]
        vnu_out[...] = vnu
        vrow_out[...] = _dotd(vnu, a1wb[...]) + a1b[...]


def _vn_stage(xp, brow, cnt, vn):
    m = vn['mlp']
    return pl.pallas_call(
        _vn_body,
        grid=(NBLK,),
        in_specs=[
            pl.BlockSpec((NB, H), lambda i: (i, 0)),
            pl.BlockSpec((1, NB), lambda i: (0, i)),
            pl.BlockSpec((NG, 1), lambda i: (0, 0)),
            pl.BlockSpec((1, H), lambda i: (0, 0)),
            pl.BlockSpec((H, 2 * H), lambda i: (0, 0)),
            pl.BlockSpec((1, 2 * H), lambda i: (0, 0)),
            pl.BlockSpec((1, 2 * H), lambda i: (0, 0)),
            pl.BlockSpec((1, 2 * H), lambda i: (0, 0)),
            pl.BlockSpec((2 * H, H), lambda i: (0, 0)),
            pl.BlockSpec((1, H), lambda i: (0, 0)),
            pl.BlockSpec((H, H), lambda i: (0, 0)),
            pl.BlockSpec((1, H), lambda i: (0, 0)),
        ],
        out_specs=[
            pl.BlockSpec((NG, H), lambda i: (0, 0)),
            pl.BlockSpec((NG, H), lambda i: (0, 0)),
        ],
        out_shape=[
            jax.ShapeDtypeStruct((NG, H), jnp.float32),
            jax.ShapeDtypeStruct((NG, H), jnp.float32),
        ],
        scratch_shapes=[pltpu.VMEM((NG, H), jnp.float32)],
    )(xp, brow, cnt, vn['emb'], m['w1'], m['b1'].reshape(1, 2 * H),
      m['g'].reshape(1, 2 * H), m['be'].reshape(1, 2 * H), m['w2'],
      m['b2'].reshape(1, H), vn['a1w'][H:, :], vn['a1b'].reshape(1, H))


def _node_body(xb, brow, vnu, vrow, a1wt, a2w, a2b, new, neb, neg, nebe,
               xt_out, cu_out, sn_out, acc):
    i = pl.program_id(0)

    @pl.when(i == 0)
    def _():
        acc[...] = jnp.zeros_like(acc)

    oh = (lax.broadcasted_iota(jnp.int32, (NG, NB), 0) == brow[...]
          ).astype(jnp.float32)
    oh2 = jnp.transpose(oh)
    valid = jnp.sum(oh2, axis=1, keepdims=True)
    vexp = _dot(oh2, vnu[...])
    t = jnp.tanh(_dotd(xb[...], a1wt[...]) + _dot(oh2, vrow[...]))
    e = jnp.exp(_dotd(t, a2w[...]) + a2b[...]) * valid
    cu_out[...] = vexp * e
    acc[...] += jnp.sum(e, axis=0, keepdims=True).sum(axis=1, keepdims=True)
    y = _dotd(xb[...], new[...]) + neb[...]
    xt_out[...] = jnp.maximum(y * (neg[...] * _BNS) + nebe[...], 0.0)

    @pl.when(i == NBLK - 1)
    def _():
        sn_out[...] = acc[...]


def _node_stage(xp, brow, vnu, vrow, vn, lp):
    return pl.pallas_call(
        _node_body,
        grid=(NBLK,),
        in_specs=[
            pl.BlockSpec((NB, H), lambda i: (i, 0)),
            pl.BlockSpec((1, NB), lambda i: (0, i)),
            pl.BlockSpec((NG, H), lambda i: (0, 0)),
            pl.BlockSpec((NG, H), lambda i: (0, 0)),
            pl.BlockSpec((H, H), lambda i: (0, 0)),
            pl.BlockSpec((H, 1), lambda i: (0, 0)),
            pl.BlockSpec((1, 1), lambda i: (0, 0)),
            pl.BlockSpec((H, H), lambda i: (0, 0)),
            pl.BlockSpec((1, H), lambda i: (0, 0)),
            pl.BlockSpec((1, H), lambda i: (0, 0)),
            pl.BlockSpec((1, H), lambda i: (0, 0)),
        ],
        out_specs=[
            pl.BlockSpec((NB, H), lambda i: (i, 0)),
            pl.BlockSpec((NB, H), lambda i: (i, 0)),
            pl.BlockSpec((1, 1), lambda i: (0, 0)),
        ],
        out_shape=[
            jax.ShapeDtypeStruct((NPAD, H), jnp.float32),
            jax.ShapeDtypeStruct((NPAD, H), jnp.float32),
            jax.ShapeDtypeStruct((1, 1), jnp.float32),
        ],
        scratch_shapes=[pltpu.VMEM((1, 1), jnp.float32)],
    )(xp, brow, vnu, vrow, vn['a1w'][:H, :], vn['a2w'],
      vn['a2b'].reshape(1, 1), lp['ne_w'], lp['ne_b'].reshape(1, H),
      lp['ne_g'].reshape(1, H), lp['ne_be'].reshape(1, H))


def _edge_body(nib, njb, eab, eew, eeb, eeg, eebe, a1wt, a1wb, a1b, a2w, a2b,
               msg_out, se_out, acc):
    i = pl.program_id(0)

    @pl.when(i == 0)
    def _():
        acc[...] = jnp.zeros_like(acc)

    ef = jnp.maximum((_dotd(eab[...], eew[...]) + eeb[...])
                     * (eeg[...] * _BNS) + eebe[...], 0.0)
    ni_ef = nib[...] + ef
    t = jnp.tanh(_dotd(ni_ef, a1wt[...]) + _dotd(njb[...], a1wb[...])
                 + a1b[...])
    lg = _dotd(t, a2w[...]) + a2b[...]
    e = jnp.exp(jnp.where(lg > 0, lg, 0.2 * lg))
    msg_out[...] = (njb[...] + ef) * e
    acc[...] += jnp.sum(e, axis=0, keepdims=True).sum(axis=1, keepdims=True)

    @pl.when(i == EBLK - 1)
    def _():
        se_out[...] = acc[...]


def _edge_stage(ni, nj, ea, lp):
    return pl.pallas_call(
        _edge_body,
        grid=(EBLK,),
        in_specs=[
            pl.BlockSpec((EB, H), lambda i: (i, 0)),
            pl.BlockSpec((EB, H), lambda i: (i, 0)),
            pl.BlockSpec((EB, 16), lambda i: (i, 0)),
            pl.BlockSpec((16, H), lambda i: (0, 0)),
            pl.BlockSpec((1, H), lambda i: (0, 0)),
            pl.BlockSpec((1, H), lambda i: (0, 0)),
            pl.BlockSpec((1, H), lambda i: (0, 0)),
            pl.BlockSpec((H, H), lambda i: (0, 0)),
            pl.BlockSpec((H, H), lambda i: (0, 0)),
            pl.BlockSpec((1, H), lambda i: (0, 0)),
            pl.BlockSpec((H, 1), lambda i: (0, 0)),
            pl.BlockSpec((1, 1), lambda i: (0, 0)),
        ],
        out_specs=[
            pl.BlockSpec((EB, H), lambda i: (i, 0)),
            pl.BlockSpec((1, 1), lambda i: (0, 0)),
        ],
        out_shape=[
            jax.ShapeDtypeStruct((NE, H), jnp.float32),
            jax.ShapeDtypeStruct((1, 1), jnp.float32),
        ],
        scratch_shapes=[pltpu.VMEM((1, 1), jnp.float32)],
    )(ni, nj, ea, lp['ee_w'], lp['ee_b'].reshape(1, H),
      lp['ee_g'].reshape(1, H), lp['ee_be'].reshape(1, H),
      lp['a1w'][:H, :], lp['a1w'][H:, :], lp['a1b'].reshape(1, H),
      lp['a2w'], lp['a2b'].reshape(1, 1))


def _combine_impl(xtb, a0b, a1b_, cub, se, sn, eps, w1, b1, g, be, w2, b2,
                  xpb, out):
    h0 = ((1.0 + eps[...]) * xtb[...]
          + (a0b[...] + a1b_[...]) * (1.0 / se[...])
          + cub[...] * (1.0 / sn[...]))
    h = jnp.maximum((_dotd(h0, w1[...]) + b1[...]) * (g[...] * _BNS)
                    + be[...], 0.0)
    o = _dotd(h, w2[...]) + b2[...]
    if xpb is not None:
        o = o + xpb[...]
    out[...] = o


def _combine_body_first(xtb, a0b, a1b_, cub, se, sn, eps, w1, b1, g, be,
                        w2, b2, out):
    _combine_impl(xtb, a0b, a1b_, cub, se, sn, eps, w1, b1, g, be, w2, b2,
                  None, out)


def _combine_stage(xt, aggp, cu, se, sn, lp, x_prev):
    m = lp['mlp']
    has_prev = x_prev is not None
    body = _combine_impl if has_prev else _combine_body_first
    in_specs = [
        pl.BlockSpec((NB, H), lambda i: (i, 0)),
        pl.BlockSpec((NB, H), lambda i: (i, 0)),
        pl.BlockSpec((NB, H), lambda i: (i + NBLK, 0)),
        pl.BlockSpec((NB, H), lambda i: (i, 0)),
        pl.BlockSpec((1, 1), lambda i: (0, 0)),
        pl.BlockSpec((1, 1), lambda i: (0, 0)),
        pl.BlockSpec((1, 1), lambda i: (0, 0)),
        pl.BlockSpec((H, 2 * H), lambda i: (0, 0)),
        pl.BlockSpec((1, 2 * H), lambda i: (0, 0)),
        pl.BlockSpec((1, 2 * H), lambda i: (0, 0)),
        pl.BlockSpec((1, 2 * H), lambda i: (0, 0)),
        pl.BlockSpec((2 * H, H), lambda i: (0, 0)),
        pl.BlockSpec((1, H), lambda i: (0, 0)),
    ]
    args = [xt, aggp, aggp, cu, se, sn, lp['eps'].reshape(1, 1),
            m['w1'], m['b1'].reshape(1, 2 * H), m['g'].reshape(1, 2 * H),
            m['be'].reshape(1, 2 * H), m['w2'], m['b2'].reshape(1, H)]
    if has_prev:
        in_specs.append(pl.BlockSpec((NB, H), lambda i: (i, 0)))
        args.append(x_prev)
    return pl.pallas_call(
        body,
        grid=(NBLK,),
        in_specs=in_specs,
        out_specs=pl.BlockSpec((NB, H), lambda i: (i, 0)),
        out_shape=jax.ShapeDtypeStruct((NPAD, H), jnp.float32),
    )(*args)


def _pool_body(xb, brow, addo, maxo, accs, accm):
    i = pl.program_id(0)

    @pl.when(i == 0)
    def _():
        accs[...] = jnp.zeros_like(accs)
        accm[...] = jnp.full_like(accm, -3e38)

    oh = (lax.broadcasted_iota(jnp.int32, (NG, NB), 0) == brow[...]
          ).astype(jnp.float32)
    accs[...] += _dot(oh, xb[...])
    masked = jnp.where(oh[:, :, None] > 0.5, xb[...][None, :, :], -3e38)
    accm[...] = jnp.maximum(accm[...], jnp.max(masked, axis=1))

    @pl.when(i == NBLK - 1)
    def _():
        addo[...] = accs[...]
        maxo[...] = accm[...]


def _pool_stage(xp, brow):
    return pl.pallas_call(
        _pool_body,
        grid=(NBLK,),
        in_specs=[
            pl.BlockSpec((NB, H), lambda i: (i, 0)),
            pl.BlockSpec((1, NB), lambda i: (0, i)),
        ],
        out_specs=[
            pl.BlockSpec((NG, H), lambda i: (0, 0)),
            pl.BlockSpec((NG, H), lambda i: (0, 0)),
        ],
        out_shape=[
            jax.ShapeDtypeStruct((NG, H), jnp.float32),
            jax.ShapeDtypeStruct((NG, H), jnp.float32),
        ],
        scratch_shapes=[pltpu.VMEM((NG, H), jnp.float32),
                        pltpu.VMEM((NG, H), jnp.float32)],
    )(xp, brow)


def _heads_body(addp, maxp, cnt, cw1, cb1, cg1, cbe1, cw2, cb2, cg2, cbe2,
                cw3, cb3, ncw1, ncb1, ncg, ncbe, ncw2, ncb2, nnw1, nnb1,
                nng, nnbe, nnw2, nnb2, fw1, fb1, fw2, fb2,
                logits, conf, emb):
    add = addp[...]
    meanp = add / cnt[...]
    emb_v = jnp.concatenate([add, meanp, maxp[...]], axis=1)
    emb[...] = emb_v
    h = jnp.maximum((_dotd(emb_v, cw1[...]) + cb1[...]) * (cg1[...] * _BNS)
                    + cbe1[...], 0.0)
    h = jnp.maximum((_dotd(h, cw2[...]) + cb2[...]) * (cg2[...] * _BNS)
                    + cbe2[...], 0.0)
    logits[...] = _dotd(h, cw3[...]) + cb3[...]
    cl = _dotd(jnp.maximum((_dotd(emb_v, ncw1[...]) + ncb1[...])
                          * (ncg[...] * _BNS) + ncbe[...], 0.0),
              ncw2[...]) + ncb2[...]
    no = _dotd(jnp.maximum((_dotd(emb_v, nnw1[...]) + nnb1[...])
                          * (nng[...] * _BNS) + nnbe[...], 0.0),
              nnw2[...]) + nnb2[...]
    comb = jnp.concatenate([cl, no], axis=1)
    f = jnp.maximum(_dotd(comb, fw1[...]) + fb1[...], 0.0)
    z = _dotd(f, fw2[...]) + fb2[...]
    conf[...] = 1.0 / (1.0 + jnp.exp(-z))


def _heads_stage(addp, maxp, cnt, clf, nf):
    PD = 3 * H
    h2, h4 = PD // 2, PD // 4

    def fullspec(shape):
        return pl.BlockSpec(shape, lambda: tuple(0 for _ in shape))

    args = [addp, maxp, cnt,
            clf['w1'], clf['b1'].reshape(1, H), clf['g1'].reshape(1, H),
            clf['be1'].reshape(1, H), clf['w2'], clf['b2'].reshape(1, H // 2),
            clf['g2'].reshape(1, H // 2), clf['be2'].reshape(1, H // 2),
            clf['w3'], clf['b3'].reshape(1, 6),
            nf['cw1'], nf['cb1'].reshape(1, h2), nf['cg'].reshape(1, h2),
            nf['cbe'].reshape(1, h2), nf['cw2'], nf['cb2'].reshape(1, h4),
            nf['nw1'], nf['nb1'].reshape(1, h2), nf['ng'].reshape(1, h2),
            nf['nbe'].reshape(1, h2), nf['nw2'], nf['nb2'].reshape(1, h4),
            nf['fw1'], nf['fb1'].reshape(1, h4), nf['fw2'],
            nf['fb2'].reshape(1, 1)]
    return pl.pallas_call(
        _heads_body,
        in_specs=[fullspec(a.shape) for a in args],
        out_specs=[fullspec((NG, 6)), fullspec((NG, 1)), fullspec((NG, PD))],
        out_shape=[
            jax.ShapeDtypeStruct((NG, 6), jnp.float32),
            jax.ShapeDtypeStruct((NG, 1), jnp.float32),
            jax.ShapeDtypeStruct((NG, PD), jnp.float32),
        ],
    )(*args)


def _sc_gather_body(xt_hbm, row_hbm, col_hbm, ni_hbm, nj_hbm, idx_v, rows_v,
                    sem):
    wid = lax.axis_index("s") * 2 + lax.axis_index("c")
    base_w = wid * PERW

    def chunk(i, idx_hbm, out_hbm):
        b = base_w + i * CH
        pltpu.sync_copy(idx_hbm.at[pl.ds(b, CH)], idx_v)
        pltpu.async_copy(xt_hbm.at[idx_v], rows_v, sem).wait()
        pltpu.sync_copy(rows_v, out_hbm.at[pl.ds(b, CH)])

    def body_r(i, carry):
        chunk(i, row_hbm, ni_hbm)
        return carry

    def body_c(i, carry):
        chunk(i, col_hbm, nj_hbm)
        return carry

    lax.fori_loop(0, NCH, body_r, 0)
    lax.fori_loop(0, NCH, body_c, 0)


def _sc_scatter_body(msgs_hbm, row_hbm, zeros_hbm, out_hbm, shared, idx_v,
                     msg_v):
    c = lax.axis_index("c")
    s = lax.axis_index("s")
    wid = s * 2 + c
    base_w = wid * PERW
    pltpu.sync_copy(zeros_hbm.at[pl.ds(s * ROWS_PER_TILE, ROWS_PER_TILE)],
                    shared.at[pl.ds(s * ROWS_PER_TILE, ROWS_PER_TILE)])
    plsc.subcore_barrier()

    def body(i, carry):
        b = base_w + i * CH
        pltpu.sync_copy(row_hbm.at[pl.ds(b, CH)], idx_v)
        pltpu.sync_copy(msgs_hbm.at[pl.ds(b, CH)], msg_v)
        pltpu.sync_copy(msg_v, shared.at[idx_v], add=True)
        return carry

    lax.fori_loop(0, NCH, body, 0)
    plsc.subcore_barrier()
    pltpu.sync_copy(
        shared.at[pl.ds(s * ROWS_PER_TILE, ROWS_PER_TILE)],
        out_hbm.at[pl.ds(c * NPAD + s * ROWS_PER_TILE, ROWS_PER_TILE)])


@functools.cache
def _sc_kernels():
    mesh = plsc.VectorSubcoreMesh(core_axis_name="c", subcore_axis_name="s")
    gather = pl.kernel(
        _sc_gather_body,
        out_type=(jax.ShapeDtypeStruct((NE, H), jnp.float32),
                  jax.ShapeDtypeStruct((NE, H), jnp.float32)),
        mesh=mesh,
        scratch_types=[pltpu.VMEM((CH,), jnp.int32),
                       pltpu.VMEM((CH, H), jnp.float32),
                       pltpu.SemaphoreType.DMA],
    )
    scatter = pl.kernel(
        _sc_scatter_body,
        out_type=jax.ShapeDtypeStruct((2 * NPAD, H), jnp.float32),
        mesh=mesh,
        scratch_types=[pltpu.VMEM_SHARED((NPAD, H), jnp.float32),
                       pltpu.VMEM((CH,), jnp.int32),
                       pltpu.VMEM((CH, H), jnp.float32)],
    )
    return gather, scatter


def _sc_gather(xt, row, col):
    return _sc_kernels()[0](xt, row, col)


def _sc_scatter(msgs, row, zeros):
    return _sc_kernels()[1](msgs, row, zeros)


def kernel(x, edge_index, edge_attr, batch, params):
    xpad = jnp.pad(_f32(x), ((0, NPAD - N), (0, 0)))
    bpad = jnp.pad(batch, (0, NPAD - N), constant_values=NG)
    brow = bpad.reshape(1, NPAD)
    row = edge_index[0]
    col = edge_index[1]
    ea = _f32(edge_attr)
    zeros_pad = jnp.zeros((NPAD, H), jnp.float32)

    xp, cnt = _encoder(xpad, brow, params)
    vn = params['vn']
    x_prev = None
    for li, lp in enumerate(params['layers']):
        vnu, vrow = _vn_stage(xp, brow, cnt, vn)
        xt, cu, sn = _node_stage(xp, brow, vnu, vrow, vn, lp)
        ni, nj = _sc_gather(xt, row, col)
        msgs, se = _edge_stage(ni, nj, ea, lp)
        aggp = _sc_scatter(msgs, row, zeros_pad)
        xp = _combine_stage(xt, aggp, cu, se, sn, lp, x_prev)
        x_prev = xp

    addp, maxp = _pool_stage(xp, brow)
    logits, conf, emb = _heads_stage(addp, maxp, cnt, params['clf'],
                                     params['nf'])
    return logits, conf, emb


# speedup vs baseline: 2.2589x; 1.3653x over previous
import functools
import math

import jax
import jax.numpy as jnp
from jax import lax
from jax.experimental import pallas as pl
from jax.experimental.pallas import tpu as pltpu
from jax.experimental.pallas import tpu_sc as plsc

H = 128
NG = 64
N = 10000
NE = 320000
NB = 128
NPAD = 10240
NBLK = NPAD // NB
EB = 512
EBLK = NE // EB
NW = 32
PERW = NE // NW
CH = 80
NCH = PERW // CH
ROWS_PER_TILE = NPAD // 16

_BNS = 1.0 / math.sqrt(1.0 + 1e-5)


def _f32(x):
    return x.astype(jnp.float32)


def _dot(a, b):
    return jax.lax.dot_general(a, b, (((1,), (0,)), ((), ())),
                               precision=jax.lax.Precision.HIGHEST,
                               preferred_element_type=jnp.float32)


def _dotd(a, b):
    return jax.lax.dot_general(a, b, (((1,), (0,)), ((), ())),
                               preferred_element_type=jnp.float32)


def _enc_body(xb, brow, iew, ieb, ieg, iebe, x0, cnt, acc):
    i = pl.program_id(0)

    @pl.when(i == 0)
    def _():
        acc[...] = jnp.zeros_like(acc)

    oh = (lax.broadcasted_iota(jnp.int32, (NG, NB), 0) == brow[...]
          ).astype(jnp.float32)
    acc[...] += jnp.sum(oh, axis=1, keepdims=True)
    y = _dotd(xb[...], iew[...]) + ieb[...]
    x0[...] = jnp.maximum(y * (ieg[...] * _BNS) + iebe[...], 0.0)

    @pl.when(i == NBLK - 1)
    def _():
        cnt[...] = jnp.maximum(acc[...], 1.0)


def _encoder(xp, brow, p):
    return pl.pallas_call(
        _enc_body,
        grid=(NBLK,),
        in_specs=[
            pl.BlockSpec((NB, H), lambda i: (i, 0)),
            pl.BlockSpec((1, NB), lambda i: (0, i)),
            pl.BlockSpec((H, H), lambda i: (0, 0)),
            pl.BlockSpec((1, H), lambda i: (0, 0)),
            pl.BlockSpec((1, H), lambda i: (0, 0)),
            pl.BlockSpec((1, H), lambda i: (0, 0)),
        ],
        out_specs=[
            pl.BlockSpec((NB, H), lambda i: (i, 0)),
            pl.BlockSpec((NG, 1), lambda i: (0, 0)),
        ],
        out_shape=[
            jax.ShapeDtypeStruct((NPAD, H), jnp.float32),
            jax.ShapeDtypeStruct((NG, 1), jnp.float32),
        ],
        scratch_shapes=[pltpu.VMEM((NG, 1), jnp.float32)],
    )(xp, brow, p['ie_w'], p['ie_b'].reshape(1, H),
      p['ie_g'].reshape(1, H), p['ie_be'].reshape(1, H))


def _vn_body(xb, brow, cnt, vemb, w1, b1, g, be, w2, b2, a1wb, a1b,
             vnu_out, vrow_out, acc):
    i = pl.program_id(0)

    @pl.when(i == 0)
    def _():
        acc[...] = jnp.zeros_like(acc)

    oh = (lax.broadcasted_iota(jnp.int32, (NG, NB), 0) == brow[...]
          ).astype(jnp.float32)
    acc[...] += _dot(oh, xb[...])

    @pl.when(i == NBLK - 1)
    def _():
        vn_in = acc[...] / cnt[...]
        z = vemb[...] + vn_in
        h = jnp.maximum((_dotd(z, w1[...]) + b1[...]) * (g[...] * _BNS)
                        + be[...], 0.0)
        vnu = _dotd(h, w2[...]) + b2[...]
        vnu_out[...] = vnu
        vrow_out[...] = _dotd(vnu, a1wb[...]) + a1b[...]


def _vn_stage(xp, brow, cnt, vn):
    m = vn['mlp']
    return pl.pallas_call(
        _vn_body,
        grid=(NBLK,),
        in_specs=[
            pl.BlockSpec((NB, H), lambda i: (i, 0)),
            pl.BlockSpec((1, NB), lambda i: (0, i)),
            pl.BlockSpec((NG, 1), lambda i: (0, 0)),
            pl.BlockSpec((1, H), lambda i: (0, 0)),
            pl.BlockSpec((H, 2 * H), lambda i: (0, 0)),
            pl.BlockSpec((1, 2 * H), lambda i: (0, 0)),
            pl.BlockSpec((1, 2 * H), lambda i: (0, 0)),
            pl.BlockSpec((1, 2 * H), lambda i: (0, 0)),
            pl.BlockSpec((2 * H, H), lambda i: (0, 0)),
            pl.BlockSpec((1, H), lambda i: (0, 0)),
            pl.BlockSpec((H, H), lambda i: (0, 0)),
            pl.BlockSpec((1, H), lambda i: (0, 0)),
        ],
        out_specs=[
            pl.BlockSpec((NG, H), lambda i: (0, 0)),
            pl.BlockSpec((NG, H), lambda i: (0, 0)),
        ],
        out_shape=[
            jax.ShapeDtypeStruct((NG, H), jnp.float32),
            jax.ShapeDtypeStruct((NG, H), jnp.float32),
        ],
        scratch_shapes=[pltpu.VMEM((NG, H), jnp.float32)],
    )(xp, brow, cnt, vn['emb'], m['w1'], m['b1'].reshape(1, 2 * H),
      m['g'].reshape(1, 2 * H), m['be'].reshape(1, 2 * H), m['w2'],
      m['b2'].reshape(1, H), vn['a1w'][H:, :], vn['a1b'].reshape(1, H))


def _node_body(xb, brow, vnu, vrow, a1wt, a2w, a2b, new, neb, neg, nebe,
               xt_out, cu_out, sn_out, acc):
    i = pl.program_id(0)

    @pl.when(i == 0)
    def _():
        acc[...] = jnp.zeros_like(acc)

    oh = (lax.broadcasted_iota(jnp.int32, (NG, NB), 0) == brow[...]
          ).astype(jnp.float32)
    oh2 = jnp.transpose(oh)
    valid = jnp.sum(oh2, axis=1, keepdims=True)
    vexp = _dot(oh2, vnu[...])
    t = jnp.tanh(_dotd(xb[...], a1wt[...]) + _dot(oh2, vrow[...]))
    e = jnp.exp(_dotd(t, a2w[...]) + a2b[...]) * valid
    cu_out[...] = vexp * e
    acc[...] += jnp.sum(e, axis=0, keepdims=True).sum(axis=1, keepdims=True)
    y = _dotd(xb[...], new[...]) + neb[...]
    xt_out[...] = jnp.maximum(y * (neg[...] * _BNS) + nebe[...], 0.0)

    @pl.when(i == NBLK - 1)
    def _():
        sn_out[...] = acc[...]


def _node_stage(xp, brow, vnu, vrow, vn, lp):
    return pl.pallas_call(
        _node_body,
        grid=(NBLK,),
        in_specs=[
            pl.BlockSpec((NB, H), lambda i: (i, 0)),
            pl.BlockSpec((1, NB), lambda i: (0, i)),
            pl.BlockSpec((NG, H), lambda i: (0, 0)),
            pl.BlockSpec((NG, H), lambda i: (0, 0)),
            pl.BlockSpec((H, H), lambda i: (0, 0)),
            pl.BlockSpec((H, 1), lambda i: (0, 0)),
            pl.BlockSpec((1, 1), lambda i: (0, 0)),
            pl.BlockSpec((H, H), lambda i: (0, 0)),
            pl.BlockSpec((1, H), lambda i: (0, 0)),
            pl.BlockSpec((1, H), lambda i: (0, 0)),
            pl.BlockSpec((1, H), lambda i: (0, 0)),
        ],
        out_specs=[
            pl.BlockSpec((NB, H), lambda i: (i, 0)),
            pl.BlockSpec((NB, H), lambda i: (i, 0)),
            pl.BlockSpec((1, 1), lambda i: (0, 0)),
        ],
        out_shape=[
            jax.ShapeDtypeStruct((NPAD, H), jnp.float32),
            jax.ShapeDtypeStruct((NPAD, H), jnp.float32),
            jax.ShapeDtypeStruct((1, 1), jnp.float32),
        ],
        scratch_shapes=[pltpu.VMEM((1, 1), jnp.float32)],
    )(xp, brow, vnu, vrow, vn['a1w'][:H, :], vn['a2w'],
      vn['a2b'].reshape(1, 1), lp['ne_w'], lp['ne_b'].reshape(1, H),
      lp['ne_g'].reshape(1, H), lp['ne_be'].reshape(1, H))


def _edge_body(nib, njb, eab, eew, eeb, eeg, eebe, a1wt, a1wb, a1b, a2w, a2b,
               msg_out, se_out, acc):
    i = pl.program_id(0)

    @pl.when(i == 0)
    def _():
        acc[...] = jnp.zeros_like(acc)

    ef = jnp.maximum((_dotd(eab[...], eew[...]) + eeb[...])
                     * (eeg[...] * _BNS) + eebe[...], 0.0)
    ni_ef = nib[...] + ef
    t = jnp.tanh(_dotd(ni_ef, a1wt[...]) + _dotd(njb[...], a1wb[...])
                 + a1b[...])
    lg = _dotd(t, a2w[...]) + a2b[...]
    e = jnp.exp(jnp.where(lg > 0, lg, 0.2 * lg))
    msg_out[...] = (njb[...] + ef) * e
    acc[...] += jnp.sum(e, axis=0, keepdims=True).sum(axis=1, keepdims=True)

    @pl.when(i == EBLK - 1)
    def _():
        se_out[...] = acc[...]


def _edge_stage(ni, nj, ea, lp):
    return pl.pallas_call(
        _edge_body,
        grid=(EBLK,),
        in_specs=[
            pl.BlockSpec((EB, H), lambda i: (i, 0)),
            pl.BlockSpec((EB, H), lambda i: (i, 0)),
            pl.BlockSpec((EB, 16), lambda i: (i, 0)),
            pl.BlockSpec((16, H), lambda i: (0, 0)),
            pl.BlockSpec((1, H), lambda i: (0, 0)),
            pl.BlockSpec((1, H), lambda i: (0, 0)),
            pl.BlockSpec((1, H), lambda i: (0, 0)),
            pl.BlockSpec((H, H), lambda i: (0, 0)),
            pl.BlockSpec((H, H), lambda i: (0, 0)),
            pl.BlockSpec((1, H), lambda i: (0, 0)),
            pl.BlockSpec((H, 1), lambda i: (0, 0)),
            pl.BlockSpec((1, 1), lambda i: (0, 0)),
        ],
        out_specs=[
            pl.BlockSpec((EB, H), lambda i: (i, 0)),
            pl.BlockSpec((1, 1), lambda i: (0, 0)),
        ],
        out_shape=[
            jax.ShapeDtypeStruct((NE, H), jnp.float32),
            jax.ShapeDtypeStruct((1, 1), jnp.float32),
        ],
        scratch_shapes=[pltpu.VMEM((1, 1), jnp.float32)],
    )(ni, nj, ea, lp['ee_w'], lp['ee_b'].reshape(1, H),
      lp['ee_g'].reshape(1, H), lp['ee_be'].reshape(1, H),
      lp['a1w'][:H, :], lp['a1w'][H:, :], lp['a1b'].reshape(1, H),
      lp['a2w'], lp['a2b'].reshape(1, 1))


def _combine_impl(xtb, a0b, a1b_, cub, se, sn, eps, w1, b1, g, be, w2, b2,
                  xpb, out):
    h0 = ((1.0 + eps[...]) * xtb[...]
          + (a0b[...] + a1b_[...]) * (1.0 / se[...])
          + cub[...] * (1.0 / sn[...]))
    h = jnp.maximum((_dotd(h0, w1[...]) + b1[...]) * (g[...] * _BNS)
                    + be[...], 0.0)
    o = _dotd(h, w2[...]) + b2[...]
    if xpb is not None:
        o = o + xpb[...]
    out[...] = o


def _combine_body_first(xtb, a0b, a1b_, cub, se, sn, eps, w1, b1, g, be,
                        w2, b2, out):
    _combine_impl(xtb, a0b, a1b_, cub, se, sn, eps, w1, b1, g, be, w2, b2,
                  None, out)


def _combine_stage(xt, aggp, cu, se, sn, lp, x_prev):
    m = lp['mlp']
    has_prev = x_prev is not None
    body = _combine_impl if has_prev else _combine_body_first
    in_specs = [
        pl.BlockSpec((NB, H), lambda i: (i, 0)),
        pl.BlockSpec((NB, H), lambda i: (i, 0)),
        pl.BlockSpec((NB, H), lambda i: (i + NBLK, 0)),
        pl.BlockSpec((NB, H), lambda i: (i, 0)),
        pl.BlockSpec((1, 1), lambda i: (0, 0)),
        pl.BlockSpec((1, 1), lambda i: (0, 0)),
        pl.BlockSpec((1, 1), lambda i: (0, 0)),
        pl.BlockSpec((H, 2 * H), lambda i: (0, 0)),
        pl.BlockSpec((1, 2 * H), lambda i: (0, 0)),
        pl.BlockSpec((1, 2 * H), lambda i: (0, 0)),
        pl.BlockSpec((1, 2 * H), lambda i: (0, 0)),
        pl.BlockSpec((2 * H, H), lambda i: (0, 0)),
        pl.BlockSpec((1, H), lambda i: (0, 0)),
    ]
    args = [xt, aggp, aggp, cu, se, sn, lp['eps'].reshape(1, 1),
            m['w1'], m['b1'].reshape(1, 2 * H), m['g'].reshape(1, 2 * H),
            m['be'].reshape(1, 2 * H), m['w2'], m['b2'].reshape(1, H)]
    if has_prev:
        in_specs.append(pl.BlockSpec((NB, H), lambda i: (i, 0)))
        args.append(x_prev)
    return pl.pallas_call(
        body,
        grid=(NBLK,),
        in_specs=in_specs,
        out_specs=pl.BlockSpec((NB, H), lambda i: (i, 0)),
        out_shape=jax.ShapeDtypeStruct((NPAD, H), jnp.float32),
    )(*args)


def _pool_body(xb, brow, addo, maxo, accs, accm):
    i = pl.program_id(0)

    @pl.when(i == 0)
    def _():
        accs[...] = jnp.zeros_like(accs)
        accm[...] = jnp.full_like(accm, -3e38)

    oh = (lax.broadcasted_iota(jnp.int32, (NG, NB), 0) == brow[...]
          ).astype(jnp.float32)
    accs[...] += _dot(oh, xb[...])
    masked = jnp.where(oh[:, :, None] > 0.5, xb[...][None, :, :], -3e38)
    accm[...] = jnp.maximum(accm[...], jnp.max(masked, axis=1))

    @pl.when(i == NBLK - 1)
    def _():
        addo[...] = accs[...]
        maxo[...] = accm[...]


def _pool_stage(xp, brow):
    return pl.pallas_call(
        _pool_body,
        grid=(NBLK,),
        in_specs=[
            pl.BlockSpec((NB, H), lambda i: (i, 0)),
            pl.BlockSpec((1, NB), lambda i: (0, i)),
        ],
        out_specs=[
            pl.BlockSpec((NG, H), lambda i: (0, 0)),
            pl.BlockSpec((NG, H), lambda i: (0, 0)),
        ],
        out_shape=[
            jax.ShapeDtypeStruct((NG, H), jnp.float32),
            jax.ShapeDtypeStruct((NG, H), jnp.float32),
        ],
        scratch_shapes=[pltpu.VMEM((NG, H), jnp.float32),
                        pltpu.VMEM((NG, H), jnp.float32)],
    )(xp, brow)


def _heads_body(addp, maxp, cnt, cw1, cb1, cg1, cbe1, cw2, cb2, cg2, cbe2,
                cw3, cb3, ncw1, ncb1, ncg, ncbe, ncw2, ncb2, nnw1, nnb1,
                nng, nnbe, nnw2, nnb2, fw1, fb1, fw2, fb2,
                logits, conf, emb):
    add = addp[...]
    meanp = add / cnt[...]
    emb_v = jnp.concatenate([add, meanp, maxp[...]], axis=1)
    emb[...] = emb_v
    h = jnp.maximum((_dotd(emb_v, cw1[...]) + cb1[...]) * (cg1[...] * _BNS)
                    + cbe1[...], 0.0)
    h = jnp.maximum((_dotd(h, cw2[...]) + cb2[...]) * (cg2[...] * _BNS)
                    + cbe2[...], 0.0)
    logits[...] = _dotd(h, cw3[...]) + cb3[...]
    cl = _dotd(jnp.maximum((_dotd(emb_v, ncw1[...]) + ncb1[...])
                          * (ncg[...] * _BNS) + ncbe[...], 0.0),
              ncw2[...]) + ncb2[...]
    no = _dotd(jnp.maximum((_dotd(emb_v, nnw1[...]) + nnb1[...])
                          * (nng[...] * _BNS) + nnbe[...], 0.0),
              nnw2[...]) + nnb2[...]
    comb = jnp.concatenate([cl, no], axis=1)
    f = jnp.maximum(_dotd(comb, fw1[...]) + fb1[...], 0.0)
    z = _dotd(f, fw2[...]) + fb2[...]
    conf[...] = 1.0 / (1.0 + jnp.exp(-z))


def _heads_stage(addp, maxp, cnt, clf, nf):
    PD = 3 * H
    h2, h4 = PD // 2, PD // 4

    def fullspec(shape):
        return pl.BlockSpec(shape, lambda: tuple(0 for _ in shape))

    args = [addp, maxp, cnt,
            clf['w1'], clf['b1'].reshape(1, H), clf['g1'].reshape(1, H),
            clf['be1'].reshape(1, H), clf['w2'], clf['b2'].reshape(1, H // 2),
            clf['g2'].reshape(1, H // 2), clf['be2'].reshape(1, H // 2),
            clf['w3'], clf['b3'].reshape(1, 6),
            nf['cw1'], nf['cb1'].reshape(1, h2), nf['cg'].reshape(1, h2),
            nf['cbe'].reshape(1, h2), nf['cw2'], nf['cb2'].reshape(1, h4),
            nf['nw1'], nf['nb1'].reshape(1, h2), nf['ng'].reshape(1, h2),
            nf['nbe'].reshape(1, h2), nf['nw2'], nf['nb2'].reshape(1, h4),
            nf['fw1'], nf['fb1'].reshape(1, h4), nf['fw2'],
            nf['fb2'].reshape(1, 1)]
    return pl.pallas_call(
        _heads_body,
        in_specs=[fullspec(a.shape) for a in args],
        out_specs=[fullspec((NG, 6)), fullspec((NG, 1)), fullspec((NG, PD))],
        out_shape=[
            jax.ShapeDtypeStruct((NG, 6), jnp.float32),
            jax.ShapeDtypeStruct((NG, 1), jnp.float32),
            jax.ShapeDtypeStruct((NG, PD), jnp.float32),
        ],
    )(*args)


CHG = 128
NCHG = PERW // CHG
GTAIL = PERW - NCHG * CHG
CHS = 80
NCHS = PERW // CHS


def _sc_gather_body(xt_hbm, row_hbm, col_hbm, ni_hbm, nj_hbm,
                    idx_r, idx_c, g0, g1, sg0, sg1, ss0, ss1):
    wid = lax.axis_index("s") * 2 + lax.axis_index("c")
    base_w = wid * PERW
    pltpu.sync_copy(row_hbm.at[wid], idx_r)
    pltpu.sync_copy(col_hbm.at[wid], idx_c)
    bufs = (g0, g1)
    gsems = (sg0, sg1)
    osems = (ss0, ss1)

    for idx, out in ((idx_r, ni_hbm), (idx_c, nj_hbm)):
        pltpu.async_copy(xt_hbm.at[idx.at[pl.ds(0, CHG)]], g0, sg0)
        pltpu.async_copy(xt_hbm.at[idx.at[pl.ds(CHG, CHG)]], g1, sg1)

        def step(i, carry):
            k0 = i * 2
            for b in range(2):
                k = k0 + b
                buf, gs, os = bufs[b], gsems[b], osems[b]
                pltpu.make_async_copy(xt_hbm.at[idx.at[pl.ds(0, CHG)]],
                                      buf, gs).wait()
                pltpu.async_copy(buf, out.at[pl.ds(base_w + k * CHG, CHG)],
                                 os)

                @pl.when(k + 2 < NCHG)
                def _():
                    pltpu.make_async_copy(
                        buf, out.at[pl.ds(base_w, CHG)], os).wait()
                    pltpu.async_copy(
                        xt_hbm.at[idx.at[pl.ds((k + 2) * CHG, CHG)]], buf, gs)
            return carry

        lax.fori_loop(0, NCHG // 2, step, 0)
        for b in range(2):
            pltpu.make_async_copy(bufs[b], out.at[pl.ds(base_w, CHG)],
                                  osems[b]).wait()
        tail = g0.at[pl.ds(0, GTAIL)]
        pltpu.async_copy(xt_hbm.at[idx.at[pl.ds(NCHG * CHG, GTAIL)]],
                         tail, sg0).wait()
        pltpu.sync_copy(tail, out.at[pl.ds(base_w + NCHG * CHG, GTAIL)])


def _sc_scatter_body(msgs_hbm, row3_hbm, zeros_hbm, out_hbm,
                     shared, idx3, m0, m1, sl0, sl1):
    c = lax.axis_index("c")
    s = lax.axis_index("s")
    wid = s * 2 + c
    base_w = wid * PERW
    pltpu.sync_copy(row3_hbm.at[wid], idx3)
    pltpu.sync_copy(zeros_hbm.at[pl.ds(s * ROWS_PER_TILE, ROWS_PER_TILE)],
                    shared.at[pl.ds(s * ROWS_PER_TILE, ROWS_PER_TILE)])
    plsc.subcore_barrier()
    bufs = (m0, m1)
    sems = (sl0, sl1)
    pltpu.async_copy(msgs_hbm.at[pl.ds(base_w, CHS)], m0, sl0)
    pltpu.async_copy(msgs_hbm.at[pl.ds(base_w + CHS, CHS)], m1, sl1)

    def step(i, carry):
        k0 = i * 2
        for b in range(2):
            k = k0 + b
            buf, sl = bufs[b], sems[b]
            pltpu.make_async_copy(msgs_hbm.at[pl.ds(base_w, CHS)],
                                  buf, sl).wait()
            pltpu.sync_copy(buf, shared.at[idx3.at[k]], add=True)

            @pl.when(k + 2 < NCHS)
            def _():
                pltpu.async_copy(
                    msgs_hbm.at[pl.ds(base_w + (k + 2) * CHS, CHS)], buf, sl)
        return carry

    lax.fori_loop(0, NCHS // 2, step, 0)
    pltpu.make_async_copy(msgs_hbm.at[pl.ds(base_w, CHS)], m0, sl0).wait()
    pltpu.sync_copy(m0, shared.at[idx3.at[NCHS - 1]], add=True)
    plsc.subcore_barrier()
    pltpu.sync_copy(
        shared.at[pl.ds(s * ROWS_PER_TILE, ROWS_PER_TILE)],
        out_hbm.at[pl.ds(c * NPAD + s * ROWS_PER_TILE, ROWS_PER_TILE)])


@functools.cache
def _sc_kernels():
    mesh = plsc.VectorSubcoreMesh(core_axis_name="c", subcore_axis_name="s")
    gather = pl.kernel(
        _sc_gather_body,
        out_type=(jax.ShapeDtypeStruct((NE, H), jnp.float32),
                  jax.ShapeDtypeStruct((NE, H), jnp.float32)),
        mesh=mesh,
        scratch_types=[pltpu.VMEM((PERW,), jnp.int32),
                       pltpu.VMEM((PERW,), jnp.int32),
                       pltpu.VMEM((CHG, H), jnp.float32),
                       pltpu.VMEM((CHG, H), jnp.float32),
                       pltpu.SemaphoreType.DMA,
                       pltpu.SemaphoreType.DMA,
                       pltpu.SemaphoreType.DMA,
                       pltpu.SemaphoreType.DMA],
    )
    scatter = pl.kernel(
        _sc_scatter_body,
        out_type=jax.ShapeDtypeStruct((2 * NPAD, H), jnp.float32),
        mesh=mesh,
        scratch_types=[pltpu.VMEM_SHARED((NPAD, H), jnp.float32),
                       pltpu.VMEM((NCHS, CHS), jnp.int32),
                       pltpu.VMEM((CHS, H), jnp.float32),
                       pltpu.VMEM((CHS, H), jnp.float32),
                       pltpu.SemaphoreType.DMA,
                       pltpu.SemaphoreType.DMA],
    )
    return gather, scatter


def _sc_gather(xt, row2, col2):
    return _sc_kernels()[0](xt, row2, col2)


def _sc_scatter(msgs, row3, zeros):
    return _sc_kernels()[1](msgs, row3, zeros)


def kernel(x, edge_index, edge_attr, batch, params):
    xpad = jnp.pad(_f32(x), ((0, NPAD - N), (0, 0)))
    bpad = jnp.pad(batch, (0, NPAD - N), constant_values=NG)
    brow = bpad.reshape(1, NPAD)
    row = edge_index[0]
    col = edge_index[1]
    row2 = row.reshape(NW, PERW)
    col2 = col.reshape(NW, PERW)
    row3 = row.reshape(NW, NCHS, CHS)
    ea = _f32(edge_attr)
    zeros_pad = jnp.zeros((NPAD, H), jnp.float32)

    xp, cnt = _encoder(xpad, brow, params)
    vn = params['vn']
    x_prev = None
    for li, lp in enumerate(params['layers']):
        vnu, vrow = _vn_stage(xp, brow, cnt, vn)
        xt, cu, sn = _node_stage(xp, brow, vnu, vrow, vn, lp)
        ni, nj = _sc_gather(xt, row2, col2)
        msgs, se = _edge_stage(ni, nj, ea, lp)
        aggp = _sc_scatter(msgs, row3, zeros_pad)
        xp = _combine_stage(xt, aggp, cu, se, sn, lp, x_prev)
        x_prev = xp

    addp, maxp = _pool_stage(xp, brow)
    logits, conf, emb = _heads_stage(addp, maxp, cnt, params['clf'],
                                     params['nf'])
    return logits, conf, emb


# speedup vs baseline: 2.8791x; 1.2745x over previous
import functools
import math

import jax
import jax.numpy as jnp
from jax import lax
from jax.experimental import pallas as pl
from jax.experimental.pallas import tpu as pltpu
from jax.experimental.pallas import tpu_sc as plsc

H = 128
NG = 64
N = 10000
NE = 320000
NB = 128
NPAD = 10240
NBLK = NPAD // NB
EB = 1280
EBLK = NE // EB
NW = 32
PERW = NE // NW
CH = 80
NCH = PERW // CH
ROWS_PER_TILE = NPAD // 16

_BNS = 1.0 / math.sqrt(1.0 + 1e-5)


def _f32(x):
    return x.astype(jnp.float32)


def _dot(a, b):
    return jax.lax.dot_general(a, b, (((1,), (0,)), ((), ())),
                               precision=jax.lax.Precision.HIGHEST,
                               preferred_element_type=jnp.float32)


def _dotd(a, b):
    return jax.lax.dot_general(a, b, (((1,), (0,)), ((), ())),
                               preferred_element_type=jnp.float32)


def _enc_body(xb, brow, iew, ieb, ieg, iebe, x0, cnt, acc):
    i = pl.program_id(0)

    @pl.when(i == 0)
    def _():
        acc[...] = jnp.zeros_like(acc)

    oh = (lax.broadcasted_iota(jnp.int32, (NG, NB), 0) == brow[...]
          ).astype(jnp.float32)
    acc[...] += jnp.sum(oh, axis=1, keepdims=True)
    y = _dotd(xb[...], iew[...]) + ieb[...]
    x0[...] = jnp.maximum(y * (ieg[...] * _BNS) + iebe[...], 0.0)

    @pl.when(i == NBLK - 1)
    def _():
        cnt[...] = jnp.maximum(acc[...], 1.0)


def _encoder(xp, brow, p):
    return pl.pallas_call(
        _enc_body,
        grid=(NBLK,),
        in_specs=[
            pl.BlockSpec((NB, H), lambda i: (i, 0)),
            pl.BlockSpec((1, NB), lambda i: (0, i)),
            pl.BlockSpec((H, H), lambda i: (0, 0)),
            pl.BlockSpec((1, H), lambda i: (0, 0)),
            pl.BlockSpec((1, H), lambda i: (0, 0)),
            pl.BlockSpec((1, H), lambda i: (0, 0)),
        ],
        out_specs=[
            pl.BlockSpec((NB, H), lambda i: (i, 0)),
            pl.BlockSpec((NG, 1), lambda i: (0, 0)),
        ],
        out_shape=[
            jax.ShapeDtypeStruct((NPAD, H), jnp.float32),
            jax.ShapeDtypeStruct((NG, 1), jnp.float32),
        ],
        scratch_shapes=[pltpu.VMEM((NG, 1), jnp.float32)],
    )(xp, brow, p['ie_w'], p['ie_b'].reshape(1, H),
      p['ie_g'].reshape(1, H), p['ie_be'].reshape(1, H))


def _vn_body(xb, brow, cnt, vemb, w1, b1, g, be, w2, b2, a1wb, a1b,
             vnu_out, vrow_out, acc):
    i = pl.program_id(0)

    @pl.when(i == 0)
    def _():
        acc[...] = jnp.zeros_like(acc)

    oh = (lax.broadcasted_iota(jnp.int32, (NG, NB), 0) == brow[...]
          ).astype(jnp.float32)
    acc[...] += _dot(oh, xb[...])

    @pl.when(i == NBLK - 1)
    def _():
        vn_in = acc[...] / cnt[...]
        z = vemb[...] + vn_in
        h = jnp.maximum((_dotd(z, w1[...]) + b1[...]) * (g[...] * _BNS)
                        + be[...], 0.0)
        vnu = _dotd(h, w2[...]) + b2[...]
        vnu_out[...] = vnu
        vrow_out[...] = _dotd(vnu, a1wb[...]) + a1b[...]


def _vn_stage(xp, brow, cnt, vn):
    m = vn['mlp']
    return pl.pallas_call(
        _vn_body,
        grid=(NBLK,),
        in_specs=[
            pl.BlockSpec((NB, H), lambda i: (i, 0)),
            pl.BlockSpec((1, NB), lambda i: (0, i)),
            pl.BlockSpec((NG, 1), lambda i: (0, 0)),
            pl.BlockSpec((1, H), lambda i: (0, 0)),
            pl.BlockSpec((H, 2 * H), lambda i: (0, 0)),
            pl.BlockSpec((1, 2 * H), lambda i: (0, 0)),
            pl.BlockSpec((1, 2 * H), lambda i: (0, 0)),
            pl.BlockSpec((1, 2 * H), lambda i: (0, 0)),
            pl.BlockSpec((2 * H, H), lambda i: (0, 0)),
            pl.BlockSpec((1, H), lambda i: (0, 0)),
            pl.BlockSpec((H, H), lambda i: (0, 0)),
            pl.BlockSpec((1, H), lambda i: (0, 0)),
        ],
        out_specs=[
            pl.BlockSpec((NG, H), lambda i: (0, 0)),
            pl.BlockSpec((NG, H), lambda i: (0, 0)),
        ],
        out_shape=[
            jax.ShapeDtypeStruct((NG, H), jnp.float32),
            jax.ShapeDtypeStruct((NG, H), jnp.float32),
        ],
        scratch_shapes=[pltpu.VMEM((NG, H), jnp.float32)],
    )(xp, brow, cnt, vn['emb'], m['w1'], m['b1'].reshape(1, 2 * H),
      m['g'].reshape(1, 2 * H), m['be'].reshape(1, 2 * H), m['w2'],
      m['b2'].reshape(1, H), vn['a1w'][H:, :], vn['a1b'].reshape(1, H))


def _node_body(xb, brow, vnu, vrow, a1wt, a2w, a2b, new, neb, neg, nebe,
               xt_out, cu_out, sn_out, acc):
    i = pl.program_id(0)

    @pl.when(i == 0)
    def _():
        acc[...] = jnp.zeros_like(acc)

    oh = (lax.broadcasted_iota(jnp.int32, (NG, NB), 0) == brow[...]
          ).astype(jnp.float32)
    oh2 = jnp.transpose(oh)
    valid = jnp.sum(oh2, axis=1, keepdims=True)
    vexp = _dot(oh2, vnu[...])
    t = jnp.tanh(_dotd(xb[...], a1wt[...]) + _dot(oh2, vrow[...]))
    e = jnp.exp(_dotd(t, a2w[...]) + a2b[...]) * valid
    cu_out[...] = vexp * e
    acc[...] += jnp.sum(e, axis=0, keepdims=True).sum(axis=1, keepdims=True)
    y = _dotd(xb[...], new[...]) + neb[...]
    xt_out[...] = jnp.maximum(y * (neg[...] * _BNS) + nebe[...], 0.0)

    @pl.when(i == NBLK - 1)
    def _():
        sn_out[...] = acc[...]


def _node_stage(xp, brow, vnu, vrow, vn, lp):
    return pl.pallas_call(
        _node_body,
        grid=(NBLK,),
        in_specs=[
            pl.BlockSpec((NB, H), lambda i: (i, 0)),
            pl.BlockSpec((1, NB), lambda i: (0, i)),
            pl.BlockSpec((NG, H), lambda i: (0, 0)),
            pl.BlockSpec((NG, H), lambda i: (0, 0)),
            pl.BlockSpec((H, H), lambda i: (0, 0)),
            pl.BlockSpec((H, 1), lambda i: (0, 0)),
            pl.BlockSpec((1, 1), lambda i: (0, 0)),
            pl.BlockSpec((H, H), lambda i: (0, 0)),
            pl.BlockSpec((1, H), lambda i: (0, 0)),
            pl.BlockSpec((1, H), lambda i: (0, 0)),
            pl.BlockSpec((1, H), lambda i: (0, 0)),
        ],
        out_specs=[
            pl.BlockSpec((NB, H), lambda i: (i, 0)),
            pl.BlockSpec((NB, H), lambda i: (i, 0)),
            pl.BlockSpec((1, 1), lambda i: (0, 0)),
        ],
        out_shape=[
            jax.ShapeDtypeStruct((NPAD, H), jnp.float32),
            jax.ShapeDtypeStruct((NPAD, H), jnp.float32),
            jax.ShapeDtypeStruct((1, 1), jnp.float32),
        ],
        scratch_shapes=[pltpu.VMEM((1, 1), jnp.float32)],
    )(xp, brow, vnu, vrow, vn['a1w'][:H, :], vn['a2w'],
      vn['a2b'].reshape(1, 1), lp['ne_w'], lp['ne_b'].reshape(1, H),
      lp['ne_g'].reshape(1, H), lp['ne_be'].reshape(1, H))


def _edge_body(nib, njb, eab, eew, eeb, eeg, eebe, a1wt, a1wb, a1b, a2w, a2b,
               msg_out, se_out, acc):
    i = pl.program_id(0)

    @pl.when(i == 0)
    def _():
        acc[...] = jnp.zeros_like(acc)

    ef = jnp.maximum((_dotd(eab[...], eew[...]) + eeb[...])
                     * (eeg[...] * _BNS) + eebe[...], 0.0)
    ni_ef = nib[...] + ef
    t = jnp.tanh(_dotd(ni_ef, a1wt[...]) + _dotd(njb[...], a1wb[...])
                 + a1b[...])
    lg = _dotd(t, a2w[...]) + a2b[...]
    e = jnp.exp(jnp.where(lg > 0, lg, 0.2 * lg))
    msg_out[...] = (njb[...] + ef) * e
    acc[...] += jnp.sum(e, axis=0, keepdims=True).sum(axis=1, keepdims=True)

    @pl.when(i == EBLK - 1)
    def _():
        se_out[...] = acc[...]


def _edge_stage(ni, nj, ea, lp):
    return pl.pallas_call(
        _edge_body,
        grid=(EBLK,),
        in_specs=[
            pl.BlockSpec((EB, H), lambda i: (i, 0)),
            pl.BlockSpec((EB, H), lambda i: (i, 0)),
            pl.BlockSpec((EB, 16), lambda i: (i, 0)),
            pl.BlockSpec((16, H), lambda i: (0, 0)),
            pl.BlockSpec((1, H), lambda i: (0, 0)),
            pl.BlockSpec((1, H), lambda i: (0, 0)),
            pl.BlockSpec((1, H), lambda i: (0, 0)),
            pl.BlockSpec((H, H), lambda i: (0, 0)),
            pl.BlockSpec((H, H), lambda i: (0, 0)),
            pl.BlockSpec((1, H), lambda i: (0, 0)),
            pl.BlockSpec((H, 1), lambda i: (0, 0)),
            pl.BlockSpec((1, 1), lambda i: (0, 0)),
        ],
        out_specs=[
            pl.BlockSpec((EB, H), lambda i: (i, 0)),
            pl.BlockSpec((1, 1), lambda i: (0, 0)),
        ],
        out_shape=[
            jax.ShapeDtypeStruct((NE, H), jnp.float32),
            jax.ShapeDtypeStruct((1, 1), jnp.float32),
        ],
        scratch_shapes=[pltpu.VMEM((1, 1), jnp.float32)],
    )(ni, nj, ea, lp['ee_w'], lp['ee_b'].reshape(1, H),
      lp['ee_g'].reshape(1, H), lp['ee_be'].reshape(1, H),
      lp['a1w'][:H, :], lp['a1w'][H:, :], lp['a1b'].reshape(1, H),
      lp['a2w'], lp['a2b'].reshape(1, 1))


def _combine_impl(xtb, a0b, a1b_, cub, se, sn, eps, w1, b1, g, be, w2, b2,
                  xpb, out):
    h0 = ((1.0 + eps[...]) * xtb[...]
          + (a0b[...] + a1b_[...]) * (1.0 / se[...])
          + cub[...] * (1.0 / sn[...]))
    h = jnp.maximum((_dotd(h0, w1[...]) + b1[...]) * (g[...] * _BNS)
                    + be[...], 0.0)
    o = _dotd(h, w2[...]) + b2[...]
    if xpb is not None:
        o = o + xpb[...]
    out[...] = o


def _combine_body_first(xtb, a0b, a1b_, cub, se, sn, eps, w1, b1, g, be,
                        w2, b2, out):
    _combine_impl(xtb, a0b, a1b_, cub, se, sn, eps, w1, b1, g, be, w2, b2,
                  None, out)


def _combine_stage(xt, aggp, cu, se, sn, lp, x_prev):
    m = lp['mlp']
    has_prev = x_prev is not None
    body = _combine_impl if has_prev else _combine_body_first
    in_specs = [
        pl.BlockSpec((NB, H), lambda i: (i, 0)),
        pl.BlockSpec((NB, H), lambda i: (i, 0)),
        pl.BlockSpec((NB, H), lambda i: (i + NBLK, 0)),
        pl.BlockSpec((NB, H), lambda i: (i, 0)),
        pl.BlockSpec((1, 1), lambda i: (0, 0)),
        pl.BlockSpec((1, 1), lambda i: (0, 0)),
        pl.BlockSpec((1, 1), lambda i: (0, 0)),
        pl.BlockSpec((H, 2 * H), lambda i: (0, 0)),
        pl.BlockSpec((1, 2 * H), lambda i: (0, 0)),
        pl.BlockSpec((1, 2 * H), lambda i: (0, 0)),
        pl.BlockSpec((1, 2 * H), lambda i: (0, 0)),
        pl.BlockSpec((2 * H, H), lambda i: (0, 0)),
        pl.BlockSpec((1, H), lambda i: (0, 0)),
    ]
    args = [xt, aggp, aggp, cu, se, sn, lp['eps'].reshape(1, 1),
            m['w1'], m['b1'].reshape(1, 2 * H), m['g'].reshape(1, 2 * H),
            m['be'].reshape(1, 2 * H), m['w2'], m['b2'].reshape(1, H)]
    if has_prev:
        in_specs.append(pl.BlockSpec((NB, H), lambda i: (i, 0)))
        args.append(x_prev)
    return pl.pallas_call(
        body,
        grid=(NBLK,),
        in_specs=in_specs,
        out_specs=pl.BlockSpec((NB, H), lambda i: (i, 0)),
        out_shape=jax.ShapeDtypeStruct((NPAD, H), jnp.float32),
    )(*args)


def _pool_body(xb, brow, addo, maxo, accs, accm):
    i = pl.program_id(0)

    @pl.when(i == 0)
    def _():
        accs[...] = jnp.zeros_like(accs)
        accm[...] = jnp.full_like(accm, -3e38)

    oh = (lax.broadcasted_iota(jnp.int32, (NG, NB), 0) == brow[...]
          ).astype(jnp.float32)
    accs[...] += _dot(oh, xb[...])
    masked = jnp.where(oh[:, :, None] > 0.5, xb[...][None, :, :], -3e38)
    accm[...] = jnp.maximum(accm[...], jnp.max(masked, axis=1))

    @pl.when(i == NBLK - 1)
    def _():
        addo[...] = accs[...]
        maxo[...] = accm[...]


def _pool_stage(xp, brow):
    return pl.pallas_call(
        _pool_body,
        grid=(NBLK,),
        in_specs=[
            pl.BlockSpec((NB, H), lambda i: (i, 0)),
            pl.BlockSpec((1, NB), lambda i: (0, i)),
        ],
        out_specs=[
            pl.BlockSpec((NG, H), lambda i: (0, 0)),
            pl.BlockSpec((NG, H), lambda i: (0, 0)),
        ],
        out_shape=[
            jax.ShapeDtypeStruct((NG, H), jnp.float32),
            jax.ShapeDtypeStruct((NG, H), jnp.float32),
        ],
        scratch_shapes=[pltpu.VMEM((NG, H), jnp.float32),
                        pltpu.VMEM((NG, H), jnp.float32)],
    )(xp, brow)


def _heads_body(addp, maxp, cnt, cw1, cb1, cg1, cbe1, cw2, cb2, cg2, cbe2,
                cw3, cb3, ncw1, ncb1, ncg, ncbe, ncw2, ncb2, nnw1, nnb1,
                nng, nnbe, nnw2, nnb2, fw1, fb1, fw2, fb2,
                logits, conf, emb):
    add = addp[...]
    meanp = add / cnt[...]
    emb_v = jnp.concatenate([add, meanp, maxp[...]], axis=1)
    emb[...] = emb_v
    h = jnp.maximum((_dotd(emb_v, cw1[...]) + cb1[...]) * (cg1[...] * _BNS)
                    + cbe1[...], 0.0)
    h = jnp.maximum((_dotd(h, cw2[...]) + cb2[...]) * (cg2[...] * _BNS)
                    + cbe2[...], 0.0)
    logits[...] = _dotd(h, cw3[...]) + cb3[...]
    cl = _dotd(jnp.maximum((_dotd(emb_v, ncw1[...]) + ncb1[...])
                          * (ncg[...] * _BNS) + ncbe[...], 0.0),
              ncw2[...]) + ncb2[...]
    no = _dotd(jnp.maximum((_dotd(emb_v, nnw1[...]) + nnb1[...])
                          * (nng[...] * _BNS) + nnbe[...], 0.0),
              nnw2[...]) + nnb2[...]
    comb = jnp.concatenate([cl, no], axis=1)
    f = jnp.maximum(_dotd(comb, fw1[...]) + fb1[...], 0.0)
    z = _dotd(f, fw2[...]) + fb2[...]
    conf[...] = 1.0 / (1.0 + jnp.exp(-z))


def _heads_stage(addp, maxp, cnt, clf, nf):
    PD = 3 * H
    h2, h4 = PD // 2, PD // 4

    def fullspec(shape):
        return pl.BlockSpec(shape, lambda: tuple(0 for _ in shape))

    args = [addp, maxp, cnt,
            clf['w1'], clf['b1'].reshape(1, H), clf['g1'].reshape(1, H),
            clf['be1'].reshape(1, H), clf['w2'], clf['b2'].reshape(1, H // 2),
            clf['g2'].reshape(1, H // 2), clf['be2'].reshape(1, H // 2),
            clf['w3'], clf['b3'].reshape(1, 6),
            nf['cw1'], nf['cb1'].reshape(1, h2), nf['cg'].reshape(1, h2),
            nf['cbe'].reshape(1, h2), nf['cw2'], nf['cb2'].reshape(1, h4),
            nf['nw1'], nf['nb1'].reshape(1, h2), nf['ng'].reshape(1, h2),
            nf['nbe'].reshape(1, h2), nf['nw2'], nf['nb2'].reshape(1, h4),
            nf['fw1'], nf['fb1'].reshape(1, h4), nf['fw2'],
            nf['fb2'].reshape(1, 1)]
    return pl.pallas_call(
        _heads_body,
        in_specs=[fullspec(a.shape) for a in args],
        out_specs=[fullspec((NG, 6)), fullspec((NG, 1)), fullspec((NG, PD))],
        out_shape=[
            jax.ShapeDtypeStruct((NG, 6), jnp.float32),
            jax.ShapeDtypeStruct((NG, 1), jnp.float32),
            jax.ShapeDtypeStruct((NG, PD), jnp.float32),
        ],
    )(*args)


CHG = 128
NCHG = PERW // CHG
GTAIL = PERW - NCHG * CHG
CHS = 80
NCHS = PERW // CHS


def _sc_gather_body(xt_hbm, row_hbm, col_hbm, ni_hbm, nj_hbm,
                    idx_r, idx_c, g0, g1, sg0, sg1, ss0, ss1):
    wid = lax.axis_index("s") * 2 + lax.axis_index("c")
    base_w = wid * PERW
    pltpu.sync_copy(row_hbm.at[wid], idx_r)
    pltpu.sync_copy(col_hbm.at[wid], idx_c)
    bufs = (g0, g1)
    gsems = (sg0, sg1)
    osems = (ss0, ss1)

    for idx, out in ((idx_r, ni_hbm), (idx_c, nj_hbm)):
        pltpu.async_copy(xt_hbm.at[idx.at[pl.ds(0, CHG)]], g0, sg0)
        pltpu.async_copy(xt_hbm.at[idx.at[pl.ds(CHG, CHG)]], g1, sg1)

        def step(i, carry):
            k0 = i * 2
            for b in range(2):
                k = k0 + b
                buf, gs, os = bufs[b], gsems[b], osems[b]
                pltpu.make_async_copy(xt_hbm.at[idx.at[pl.ds(0, CHG)]],
                                      buf, gs).wait()
                pltpu.async_copy(buf, out.at[pl.ds(base_w + k * CHG, CHG)],
                                 os)

                @pl.when(k + 2 < NCHG)
                def _():
                    pltpu.make_async_copy(
                        buf, out.at[pl.ds(base_w, CHG)], os).wait()
                    pltpu.async_copy(
                        xt_hbm.at[idx.at[pl.ds((k + 2) * CHG, CHG)]], buf, gs)
            return carry

        lax.fori_loop(0, NCHG // 2, step, 0)
        for b in range(2):
            pltpu.make_async_copy(bufs[b], out.at[pl.ds(base_w, CHG)],
                                  osems[b]).wait()
        tail = g0.at[pl.ds(0, GTAIL)]
        pltpu.async_copy(xt_hbm.at[idx.at[pl.ds(NCHG * CHG, GTAIL)]],
                         tail, sg0).wait()
        pltpu.sync_copy(tail, out.at[pl.ds(base_w + NCHG * CHG, GTAIL)])


def _sc_scatter_body(msgs_hbm, row3_hbm, zeros_hbm, out_hbm,
                     shared, idx3, m0, m1, sl0, sl1, sa0, sa1):
    c = lax.axis_index("c")
    s = lax.axis_index("s")
    wid = s * 2 + c
    base_w = wid * PERW
    pltpu.sync_copy(row3_hbm.at[wid], idx3)
    pltpu.sync_copy(zeros_hbm.at[pl.ds(s * ROWS_PER_TILE, ROWS_PER_TILE)],
                    shared.at[pl.ds(s * ROWS_PER_TILE, ROWS_PER_TILE)])
    plsc.subcore_barrier()
    bufs = (m0, m1)
    lsems = (sl0, sl1)
    asems = (sa0, sa1)
    pltpu.async_copy(msgs_hbm.at[pl.ds(base_w, CHS)], m0, sl0)
    pltpu.async_copy(msgs_hbm.at[pl.ds(base_w + CHS, CHS)], m1, sl1)

    def step(i, carry):
        k0 = i * 2
        for b in range(2):
            k = k0 + b
            buf, sl, sa = bufs[b], lsems[b], asems[b]
            pltpu.make_async_copy(msgs_hbm.at[pl.ds(base_w, CHS)],
                                  buf, sl).wait()
            pltpu.async_copy(buf, shared.at[idx3.at[k]], sa, add=True)

            @pl.when(k + 2 < NCHS)
            def _():
                pltpu.make_async_copy(buf, shared.at[idx3.at[k]], sa).wait()
                pltpu.async_copy(
                    msgs_hbm.at[pl.ds(base_w + (k + 2) * CHS, CHS)], buf, sl)
        return carry

    lax.fori_loop(0, NCHS // 2, step, 0)
    pltpu.make_async_copy(msgs_hbm.at[pl.ds(base_w, CHS)], m0, sl0).wait()
    pltpu.async_copy(m0, shared.at[idx3.at[NCHS - 1]], sa0, add=True)
    pltpu.make_async_copy(m0, shared.at[idx3.at[0]], sa0).wait()
    pltpu.make_async_copy(m1, shared.at[idx3.at[0]], sa1).wait()
    plsc.subcore_barrier()
    pltpu.sync_copy(
        shared.at[pl.ds(s * ROWS_PER_TILE, ROWS_PER_TILE)],
        out_hbm.at[pl.ds(c * NPAD + s * ROWS_PER_TILE, ROWS_PER_TILE)])


@functools.cache
def _sc_kernels():
    mesh = plsc.VectorSubcoreMesh(core_axis_name="c", subcore_axis_name="s")
    gather = pl.kernel(
        _sc_gather_body,
        out_type=(jax.ShapeDtypeStruct((NE, H), jnp.float32),
                  jax.ShapeDtypeStruct((NE, H), jnp.float32)),
        mesh=mesh,
        scratch_types=[pltpu.VMEM((PERW,), jnp.int32),
                       pltpu.VMEM((PERW,), jnp.int32),
                       pltpu.VMEM((CHG, H), jnp.float32),
                       pltpu.VMEM((CHG, H), jnp.float32),
                       pltpu.SemaphoreType.DMA,
                       pltpu.SemaphoreType.DMA,
                       pltpu.SemaphoreType.DMA,
                       pltpu.SemaphoreType.DMA],
    )
    scatter = pl.kernel(
        _sc_scatter_body,
        out_type=jax.ShapeDtypeStruct((2 * NPAD, H), jnp.float32),
        mesh=mesh,
        scratch_types=[pltpu.VMEM_SHARED((NPAD, H), jnp.float32),
                       pltpu.VMEM((NCHS, CHS), jnp.int32),
                       pltpu.VMEM((CHS, H), jnp.float32),
                       pltpu.VMEM((CHS, H), jnp.float32),
                       pltpu.SemaphoreType.DMA,
                       pltpu.SemaphoreType.DMA,
                       pltpu.SemaphoreType.DMA,
                       pltpu.SemaphoreType.DMA],
    )
    return gather, scatter


def _sc_gather(xt, row2, col2):
    return _sc_kernels()[0](xt, row2, col2)


def _sc_scatter(msgs, row3, zeros):
    return _sc_kernels()[1](msgs, row3, zeros)


def kernel(x, edge_index, edge_attr, batch, params):
    xpad = jnp.pad(_f32(x), ((0, NPAD - N), (0, 0)))
    bpad = jnp.pad(batch, (0, NPAD - N), constant_values=NG)
    brow = bpad.reshape(1, NPAD)
    row = edge_index[0]
    col = edge_index[1]
    row2 = row.reshape(NW, PERW)
    col2 = col.reshape(NW, PERW)
    row3 = row.reshape(NW, NCHS, CHS)
    ea = _f32(edge_attr)
    zeros_pad = jnp.zeros((NPAD, H), jnp.float32)

    xp, cnt = _encoder(xpad, brow, params)
    vn = params['vn']
    x_prev = None
    for li, lp in enumerate(params['layers']):
        vnu, vrow = _vn_stage(xp, brow, cnt, vn)
        xt, cu, sn = _node_stage(xp, brow, vnu, vrow, vn, lp)
        ni, nj = _sc_gather(xt, row2, col2)
        msgs, se = _edge_stage(ni, nj, ea, lp)
        aggp = _sc_scatter(msgs, row3, zeros_pad)
        xp = _combine_stage(xt, aggp, cu, se, sn, lp, x_prev)
        x_prev = xp

    addp, maxp = _pool_stage(xp, brow)
    logits, conf, emb = _heads_stage(addp, maxp, cnt, params['clf'],
                                     params['nf'])
    return logits, conf, emb


# speedup vs baseline: 2.8947x; 1.0054x over previous
import functools
import math

import jax
import jax.numpy as jnp
from jax import lax
from jax.experimental import pallas as pl
from jax.experimental.pallas import tpu as pltpu
from jax.experimental.pallas import tpu_sc as plsc

H = 128
NG = 64
N = 10000
NE = 320000
NB = 128
NPAD = 10240
NBLK = NPAD // NB
EB = 1280
EBLK = NE // EB
NW = 32
PERW = NE // NW
CH = 80
NCH = PERW // CH
ROWS_PER_TILE = NPAD // 16

_BNS = 1.0 / math.sqrt(1.0 + 1e-5)


def _f32(x):
    return x.astype(jnp.float32)


def _dot(a, b):
    return jax.lax.dot_general(a, b, (((1,), (0,)), ((), ())),
                               precision=jax.lax.Precision.HIGHEST,
                               preferred_element_type=jnp.float32)


def _dotd(a, b):
    return jax.lax.dot_general(a, b, (((1,), (0,)), ((), ())),
                               preferred_element_type=jnp.float32)


def _enc_body(xb, brow, iew, ieb, ieg, iebe, x0, cnt, acc):
    i = pl.program_id(0)

    @pl.when(i == 0)
    def _():
        acc[...] = jnp.zeros_like(acc)

    oh = (lax.broadcasted_iota(jnp.int32, (NG, NB), 0) == brow[...]
          ).astype(jnp.float32)
    acc[...] += jnp.sum(oh, axis=1, keepdims=True)
    y = _dotd(xb[...], iew[...]) + ieb[...]
    x0[...] = jnp.maximum(y * (ieg[...] * _BNS) + iebe[...], 0.0)

    @pl.when(i == NBLK - 1)
    def _():
        cnt[...] = jnp.maximum(acc[...], 1.0)


def _encoder(xp, brow, p):
    return pl.pallas_call(
        _enc_body,
        grid=(NBLK,),
        in_specs=[
            pl.BlockSpec((NB, H), lambda i: (i, 0)),
            pl.BlockSpec((1, NB), lambda i: (0, i)),
            pl.BlockSpec((H, H), lambda i: (0, 0)),
            pl.BlockSpec((1, H), lambda i: (0, 0)),
            pl.BlockSpec((1, H), lambda i: (0, 0)),
            pl.BlockSpec((1, H), lambda i: (0, 0)),
        ],
        out_specs=[
            pl.BlockSpec((NB, H), lambda i: (i, 0)),
            pl.BlockSpec((NG, 1), lambda i: (0, 0)),
        ],
        out_shape=[
            jax.ShapeDtypeStruct((NPAD, H), jnp.float32),
            jax.ShapeDtypeStruct((NG, 1), jnp.float32),
        ],
        scratch_shapes=[pltpu.VMEM((NG, 1), jnp.float32)],
    )(xp, brow, p['ie_w'], p['ie_b'].reshape(1, H),
      p['ie_g'].reshape(1, H), p['ie_be'].reshape(1, H))


def _vn_body(xb, brow, cnt, vemb, w1, b1, g, be, w2, b2, a1wb, a1b,
             vnu_out, vrow_out, acc):
    i = pl.program_id(0)

    @pl.when(i == 0)
    def _():
        acc[...] = jnp.zeros_like(acc)

    oh = (lax.broadcasted_iota(jnp.int32, (NG, NB), 0) == brow[...]
          ).astype(jnp.float32)
    acc[...] += _dot(oh, xb[...])

    @pl.when(i == NBLK - 1)
    def _():
        vn_in = acc[...] / cnt[...]
        z = vemb[...] + vn_in
        h = jnp.maximum((_dotd(z, w1[...]) + b1[...]) * (g[...] * _BNS)
                        + be[...], 0.0)
        vnu = _dotd(h, w2[...]) + b2[...]
        vnu_out[...] = vnu
        vrow_out[...] = _dotd(vnu, a1wb[...]) + a1b[...]


def _vn_stage(xp, brow, cnt, vn):
    m = vn['mlp']
    return pl.pallas_call(
        _vn_body,
        grid=(NBLK,),
        in_specs=[
            pl.BlockSpec((NB, H), lambda i: (i, 0)),
            pl.BlockSpec((1, NB), lambda i: (0, i)),
            pl.BlockSpec((NG, 1), lambda i: (0, 0)),
            pl.BlockSpec((1, H), lambda i: (0, 0)),
            pl.BlockSpec((H, 2 * H), lambda i: (0, 0)),
            pl.BlockSpec((1, 2 * H), lambda i: (0, 0)),
            pl.BlockSpec((1, 2 * H), lambda i: (0, 0)),
            pl.BlockSpec((1, 2 * H), lambda i: (0, 0)),
            pl.BlockSpec((2 * H, H), lambda i: (0, 0)),
            pl.BlockSpec((1, H), lambda i: (0, 0)),
            pl.BlockSpec((H, H), lambda i: (0, 0)),
            pl.BlockSpec((1, H), lambda i: (0, 0)),
        ],
        out_specs=[
            pl.BlockSpec((NG, H), lambda i: (0, 0)),
            pl.BlockSpec((NG, H), lambda i: (0, 0)),
        ],
        out_shape=[
            jax.ShapeDtypeStruct((NG, H), jnp.float32),
            jax.ShapeDtypeStruct((NG, H), jnp.float32),
        ],
        scratch_shapes=[pltpu.VMEM((NG, H), jnp.float32)],
    )(xp, brow, cnt, vn['emb'], m['w1'], m['b1'].reshape(1, 2 * H),
      m['g'].reshape(1, 2 * H), m['be'].reshape(1, 2 * H), m['w2'],
      m['b2'].reshape(1, H), vn['a1w'][H:, :], vn['a1b'].reshape(1, H))


def _node_body(xb, brow, vnu, vrow, a1wt, a2w, a2b, new, neb, neg, nebe,
               xt_out, cu_out, sn_out, acc):
    i = pl.program_id(0)

    @pl.when(i == 0)
    def _():
        acc[...] = jnp.zeros_like(acc)

    oh = (lax.broadcasted_iota(jnp.int32, (NG, NB), 0) == brow[...]
          ).astype(jnp.float32)
    oh2 = jnp.transpose(oh)
    valid = jnp.sum(oh2, axis=1, keepdims=True)
    vexp = _dot(oh2, vnu[...])
    t = jnp.tanh(_dotd(xb[...], a1wt[...]) + _dot(oh2, vrow[...]))
    e = jnp.exp(_dotd(t, a2w[...]) + a2b[...]) * valid
    cu_out[...] = vexp * e
    acc[...] += jnp.sum(e, axis=0, keepdims=True).sum(axis=1, keepdims=True)
    y = _dotd(xb[...], new[...]) + neb[...]
    xt_out[...] = jnp.maximum(y * (neg[...] * _BNS) + nebe[...], 0.0)

    @pl.when(i == NBLK - 1)
    def _():
        sn_out[...] = acc[...]


def _node_stage(xp, brow, vnu, vrow, vn, lp):
    return pl.pallas_call(
        _node_body,
        grid=(NBLK,),
        in_specs=[
            pl.BlockSpec((NB, H), lambda i: (i, 0)),
            pl.BlockSpec((1, NB), lambda i: (0, i)),
            pl.BlockSpec((NG, H), lambda i: (0, 0)),
            pl.BlockSpec((NG, H), lambda i: (0, 0)),
            pl.BlockSpec((H, H), lambda i: (0, 0)),
            pl.BlockSpec((H, 1), lambda i: (0, 0)),
            pl.BlockSpec((1, 1), lambda i: (0, 0)),
            pl.BlockSpec((H, H), lambda i: (0, 0)),
            pl.BlockSpec((1, H), lambda i: (0, 0)),
            pl.BlockSpec((1, H), lambda i: (0, 0)),
            pl.BlockSpec((1, H), lambda i: (0, 0)),
        ],
        out_specs=[
            pl.BlockSpec((NB, H), lambda i: (i, 0)),
            pl.BlockSpec((NB, H), lambda i: (i, 0)),
            pl.BlockSpec((1, 1), lambda i: (0, 0)),
        ],
        out_shape=[
            jax.ShapeDtypeStruct((NPAD, H), jnp.float32),
            jax.ShapeDtypeStruct((NPAD, H), jnp.float32),
            jax.ShapeDtypeStruct((1, 1), jnp.float32),
        ],
        scratch_shapes=[pltpu.VMEM((1, 1), jnp.float32)],
    )(xp, brow, vnu, vrow, vn['a1w'][:H, :], vn['a2w'],
      vn['a2b'].reshape(1, 1), lp['ne_w'], lp['ne_b'].reshape(1, H),
      lp['ne_g'].reshape(1, H), lp['ne_be'].reshape(1, H))


def _edge_body(nib, njb, eab, eew, eeb, eeg, eebe, a1wt, a1wb, a1b, a2w, a2b,
               msg_out, se_out, acc):
    i = pl.program_id(0)

    @pl.when(i == 0)
    def _():
        acc[...] = jnp.zeros_like(acc)

    ef = jnp.maximum((_dotd(eab[...], eew[...]) + eeb[...])
                     * (eeg[...] * _BNS) + eebe[...], 0.0)
    ni_ef = nib[...] + ef
    t = jnp.tanh(_dotd(ni_ef, a1wt[...]) + _dotd(njb[...], a1wb[...])
                 + a1b[...])
    lg = _dotd(t, a2w[...]) + a2b[...]
    e = jnp.exp(jnp.where(lg > 0, lg, 0.2 * lg))
    msg_out[...] = (njb[...] + ef) * e
    acc[...] += jnp.sum(e, axis=0, keepdims=True).sum(axis=1, keepdims=True)

    @pl.when(i == EBLK - 1)
    def _():
        se_out[...] = acc[...]


def _edge_stage(ni, nj, ea, lp):
    return pl.pallas_call(
        _edge_body,
        grid=(EBLK,),
        in_specs=[
            pl.BlockSpec((EB, H), lambda i: (i, 0)),
            pl.BlockSpec((EB, H), lambda i: (i, 0)),
            pl.BlockSpec((EB, 16), lambda i: (i, 0)),
            pl.BlockSpec((16, H), lambda i: (0, 0)),
            pl.BlockSpec((1, H), lambda i: (0, 0)),
            pl.BlockSpec((1, H), lambda i: (0, 0)),
            pl.BlockSpec((1, H), lambda i: (0, 0)),
            pl.BlockSpec((H, H), lambda i: (0, 0)),
            pl.BlockSpec((H, H), lambda i: (0, 0)),
            pl.BlockSpec((1, H), lambda i: (0, 0)),
            pl.BlockSpec((H, 1), lambda i: (0, 0)),
            pl.BlockSpec((1, 1), lambda i: (0, 0)),
        ],
        out_specs=[
            pl.BlockSpec((EB, H), lambda i: (i, 0)),
            pl.BlockSpec((1, 1), lambda i: (0, 0)),
        ],
        out_shape=[
            jax.ShapeDtypeStruct((NE, H), jnp.float32),
            jax.ShapeDtypeStruct((1, 1), jnp.float32),
        ],
        scratch_shapes=[pltpu.VMEM((1, 1), jnp.float32)],
    )(ni, nj, ea, lp['ee_w'], lp['ee_b'].reshape(1, H),
      lp['ee_g'].reshape(1, H), lp['ee_be'].reshape(1, H),
      lp['a1w'][:H, :], lp['a1w'][H:, :], lp['a1b'].reshape(1, H),
      lp['a2w'], lp['a2b'].reshape(1, 1))


def _combine_impl(xtb, a0b, a1b_, cub, se, sn, eps, w1, b1, g, be, w2, b2,
                  xpb, out):
    h0 = ((1.0 + eps[...]) * xtb[...]
          + (a0b[...] + a1b_[...]) * (1.0 / se[...])
          + cub[...] * (1.0 / sn[...]))
    h = jnp.maximum((_dotd(h0, w1[...]) + b1[...]) * (g[...] * _BNS)
                    + be[...], 0.0)
    o = _dotd(h, w2[...]) + b2[...]
    if xpb is not None:
        o = o + xpb[...]
    out[...] = o


def _combine_body_first(xtb, a0b, a1b_, cub, se, sn, eps, w1, b1, g, be,
                        w2, b2, out):
    _combine_impl(xtb, a0b, a1b_, cub, se, sn, eps, w1, b1, g, be, w2, b2,
                  None, out)


def _combine_stage(xt, aggp, cu, se, sn, lp, x_prev):
    m = lp['mlp']
    has_prev = x_prev is not None
    body = _combine_impl if has_prev else _combine_body_first
    in_specs = [
        pl.BlockSpec((NB, H), lambda i: (i, 0)),
        pl.BlockSpec((NB, H), lambda i: (i, 0)),
        pl.BlockSpec((NB, H), lambda i: (i + NBLK, 0)),
        pl.BlockSpec((NB, H), lambda i: (i, 0)),
        pl.BlockSpec((1, 1), lambda i: (0, 0)),
        pl.BlockSpec((1, 1), lambda i: (0, 0)),
        pl.BlockSpec((1, 1), lambda i: (0, 0)),
        pl.BlockSpec((H, 2 * H), lambda i: (0, 0)),
        pl.BlockSpec((1, 2 * H), lambda i: (0, 0)),
        pl.BlockSpec((1, 2 * H), lambda i: (0, 0)),
        pl.BlockSpec((1, 2 * H), lambda i: (0, 0)),
        pl.BlockSpec((2 * H, H), lambda i: (0, 0)),
        pl.BlockSpec((1, H), lambda i: (0, 0)),
    ]
    args = [xt, aggp, aggp, cu, se, sn, lp['eps'].reshape(1, 1),
            m['w1'], m['b1'].reshape(1, 2 * H), m['g'].reshape(1, 2 * H),
            m['be'].reshape(1, 2 * H), m['w2'], m['b2'].reshape(1, H)]
    if has_prev:
        in_specs.append(pl.BlockSpec((NB, H), lambda i: (i, 0)))
        args.append(x_prev)
    return pl.pallas_call(
        body,
        grid=(NBLK,),
        in_specs=in_specs,
        out_specs=pl.BlockSpec((NB, H), lambda i: (i, 0)),
        out_shape=jax.ShapeDtypeStruct((NPAD, H), jnp.float32),
    )(*args)


def _pool_body(xb, brow, addo, maxo, accs, accm):
    i = pl.program_id(0)

    @pl.when(i == 0)
    def _():
        accs[...] = jnp.zeros_like(accs)
        accm[...] = jnp.full_like(accm, -3e38)

    oh = (lax.broadcasted_iota(jnp.int32, (NG, NB), 0) == brow[...]
          ).astype(jnp.float32)
    accs[...] += _dot(oh, xb[...])
    masked = jnp.where(oh[:, :, None] > 0.5, xb[...][None, :, :], -3e38)
    accm[...] = jnp.maximum(accm[...], jnp.max(masked, axis=1))

    @pl.when(i == NBLK - 1)
    def _():
        addo[...] = accs[...]
        maxo[...] = accm[...]


def _pool_stage(xp, brow):
    return pl.pallas_call(
        _pool_body,
        grid=(NBLK,),
        in_specs=[
            pl.BlockSpec((NB, H), lambda i: (i, 0)),
            pl.BlockSpec((1, NB), lambda i: (0, i)),
        ],
        out_specs=[
            pl.BlockSpec((NG, H), lambda i: (0, 0)),
            pl.BlockSpec((NG, H), lambda i: (0, 0)),
        ],
        out_shape=[
            jax.ShapeDtypeStruct((NG, H), jnp.float32),
            jax.ShapeDtypeStruct((NG, H), jnp.float32),
        ],
        scratch_shapes=[pltpu.VMEM((NG, H), jnp.float32),
                        pltpu.VMEM((NG, H), jnp.float32)],
    )(xp, brow)


def _heads_body(addp, maxp, cnt, cw1, cb1, cg1, cbe1, cw2, cb2, cg2, cbe2,
                cw3, cb3, ncw1, ncb1, ncg, ncbe, ncw2, ncb2, nnw1, nnb1,
                nng, nnbe, nnw2, nnb2, fw1, fb1, fw2, fb2,
                logits, conf, emb):
    add = addp[...]
    meanp = add / cnt[...]
    emb_v = jnp.concatenate([add, meanp, maxp[...]], axis=1)
    emb[...] = emb_v
    h = jnp.maximum((_dotd(emb_v, cw1[...]) + cb1[...]) * (cg1[...] * _BNS)
                    + cbe1[...], 0.0)
    h = jnp.maximum((_dotd(h, cw2[...]) + cb2[...]) * (cg2[...] * _BNS)
                    + cbe2[...], 0.0)
    logits[...] = _dotd(h, cw3[...]) + cb3[...]
    cl = _dotd(jnp.maximum((_dotd(emb_v, ncw1[...]) + ncb1[...])
                          * (ncg[...] * _BNS) + ncbe[...], 0.0),
              ncw2[...]) + ncb2[...]
    no = _dotd(jnp.maximum((_dotd(emb_v, nnw1[...]) + nnb1[...])
                          * (nng[...] * _BNS) + nnbe[...], 0.0),
              nnw2[...]) + nnb2[...]
    comb = jnp.concatenate([cl, no], axis=1)
    f = jnp.maximum(_dotd(comb, fw1[...]) + fb1[...], 0.0)
    z = _dotd(f, fw2[...]) + fb2[...]
    conf[...] = 1.0 / (1.0 + jnp.exp(-z))


def _heads_stage(addp, maxp, cnt, clf, nf):
    PD = 3 * H
    h2, h4 = PD // 2, PD // 4

    def fullspec(shape):
        return pl.BlockSpec(shape, lambda: tuple(0 for _ in shape))

    args = [addp, maxp, cnt,
            clf['w1'], clf['b1'].reshape(1, H), clf['g1'].reshape(1, H),
            clf['be1'].reshape(1, H), clf['w2'], clf['b2'].reshape(1, H // 2),
            clf['g2'].reshape(1, H // 2), clf['be2'].reshape(1, H // 2),
            clf['w3'], clf['b3'].reshape(1, 6),
            nf['cw1'], nf['cb1'].reshape(1, h2), nf['cg'].reshape(1, h2),
            nf['cbe'].reshape(1, h2), nf['cw2'], nf['cb2'].reshape(1, h4),
            nf['nw1'], nf['nb1'].reshape(1, h2), nf['ng'].reshape(1, h2),
            nf['nbe'].reshape(1, h2), nf['nw2'], nf['nb2'].reshape(1, h4),
            nf['fw1'], nf['fb1'].reshape(1, h4), nf['fw2'],
            nf['fb2'].reshape(1, 1)]
    return pl.pallas_call(
        _heads_body,
        in_specs=[fullspec(a.shape) for a in args],
        out_specs=[fullspec((NG, 6)), fullspec((NG, 1)), fullspec((NG, PD))],
        out_shape=[
            jax.ShapeDtypeStruct((NG, 6), jnp.float32),
            jax.ShapeDtypeStruct((NG, 1), jnp.float32),
            jax.ShapeDtypeStruct((NG, PD), jnp.float32),
        ],
    )(*args)


CHG = 128
NCHG = PERW // CHG
GTAIL = PERW - NCHG * CHG
CHS = 80
NCHS = PERW // CHS


def _sc_gather_body(xt_hbm, row_hbm, col_hbm, ni_hbm, nj_hbm,
                    idx_r, idx_c, g0, g1, g2, g3,
                    sg0, sg1, sg2, sg3, ss0, ss1, ss2, ss3):
    wid = lax.axis_index("s") * 2 + lax.axis_index("c")
    base_w = wid * PERW
    pltpu.sync_copy(row_hbm.at[wid], idx_r)
    pltpu.sync_copy(col_hbm.at[wid], idx_c)
    phases = ((idx_r, ni_hbm, (g0, g1), (sg0, sg1), (ss0, ss1)),
              (idx_c, nj_hbm, (g2, g3), (sg2, sg3), (ss2, ss3)))
    for idx, out, bufs, gsems, osems in phases:
        pltpu.async_copy(xt_hbm.at[idx.at[pl.ds(0, CHG)]], bufs[0], gsems[0])
        pltpu.async_copy(xt_hbm.at[idx.at[pl.ds(CHG, CHG)]], bufs[1],
                         gsems[1])

    def step(i, carry):
        k0 = i * 2
        for idx, out, bufs, gsems, osems in phases:
            for b in range(2):
                k = k0 + b
                buf, gs, os = bufs[b], gsems[b], osems[b]
                pltpu.make_async_copy(xt_hbm.at[idx.at[pl.ds(0, CHG)]],
                                      buf, gs).wait()
                pltpu.async_copy(buf, out.at[pl.ds(base_w + k * CHG, CHG)],
                                 os)

                @pl.when(k + 2 < NCHG)
                def _():
                    pltpu.make_async_copy(
                        buf, out.at[pl.ds(base_w, CHG)], os).wait()
                    pltpu.async_copy(
                        xt_hbm.at[idx.at[pl.ds((k + 2) * CHG, CHG)]], buf, gs)
        return carry

    lax.fori_loop(0, NCHG // 2, step, 0)
    for idx, out, bufs, gsems, osems in phases:
        for b in range(2):
            pltpu.make_async_copy(bufs[b], out.at[pl.ds(base_w, CHG)],
                                  osems[b]).wait()
        tail = bufs[0].at[pl.ds(0, GTAIL)]
        pltpu.async_copy(xt_hbm.at[idx.at[pl.ds(NCHG * CHG, GTAIL)]],
                         tail, gsems[0]).wait()
        pltpu.sync_copy(tail, out.at[pl.ds(base_w + NCHG * CHG, GTAIL)])


def _sc_scatter_body(msgs_hbm, row3_hbm, zeros_hbm, out_hbm,
                     shared, idx3, m0, m1, sl0, sl1, sa0, sa1):
    c = lax.axis_index("c")
    s = lax.axis_index("s")
    wid = s * 2 + c
    base_w = wid * PERW
    pltpu.sync_copy(row3_hbm.at[wid], idx3)
    pltpu.sync_copy(zeros_hbm.at[pl.ds(s * ROWS_PER_TILE, ROWS_PER_TILE)],
                    shared.at[pl.ds(s * ROWS_PER_TILE, ROWS_PER_TILE)])
    plsc.subcore_barrier()
    bufs = (m0, m1)
    lsems = (sl0, sl1)
    asems = (sa0, sa1)
    pltpu.async_copy(msgs_hbm.at[pl.ds(base_w, CHS)], m0, sl0)
    pltpu.async_copy(msgs_hbm.at[pl.ds(base_w + CHS, CHS)], m1, sl1)

    def step(i, carry):
        k0 = i * 2
        for b in range(2):
            k = k0 + b
            buf, sl, sa = bufs[b], lsems[b], asems[b]
            pltpu.make_async_copy(msgs_hbm.at[pl.ds(base_w, CHS)],
                                  buf, sl).wait()
            pltpu.async_copy(buf, shared.at[idx3.at[k]], sa, add=True)

            @pl.when(k + 2 < NCHS)
            def _():
                pltpu.make_async_copy(buf, shared.at[idx3.at[k]], sa).wait()
                pltpu.async_copy(
                    msgs_hbm.at[pl.ds(base_w + (k + 2) * CHS, CHS)], buf, sl)
        return carry

    lax.fori_loop(0, NCHS // 2, step, 0)
    pltpu.make_async_copy(msgs_hbm.at[pl.ds(base_w, CHS)], m0, sl0).wait()
    pltpu.async_copy(m0, shared.at[idx3.at[NCHS - 1]], sa0, add=True)
    pltpu.make_async_copy(m0, shared.at[idx3.at[0]], sa0).wait()
    pltpu.make_async_copy(m1, shared.at[idx3.at[0]], sa1).wait()
    plsc.subcore_barrier()
    pltpu.sync_copy(
        shared.at[pl.ds(s * ROWS_PER_TILE, ROWS_PER_TILE)],
        out_hbm.at[pl.ds(c * NPAD + s * ROWS_PER_TILE, ROWS_PER_TILE)])


@functools.cache
def _sc_kernels():
    mesh = plsc.VectorSubcoreMesh(core_axis_name="c", subcore_axis_name="s")
    gather = pl.kernel(
        _sc_gather_body,
        out_type=(jax.ShapeDtypeStruct((NE, H), jnp.float32),
                  jax.ShapeDtypeStruct((NE, H), jnp.float32)),
        mesh=mesh,
        scratch_types=[pltpu.VMEM((PERW,), jnp.int32),
                       pltpu.VMEM((PERW,), jnp.int32)]
                      + [pltpu.VMEM((CHG, H), jnp.float32)] * 4
                      + [pltpu.SemaphoreType.DMA] * 8,
    )
    scatter = pl.kernel(
        _sc_scatter_body,
        out_type=jax.ShapeDtypeStruct((2 * NPAD, H), jnp.float32),
        mesh=mesh,
        scratch_types=[pltpu.VMEM_SHARED((NPAD, H), jnp.float32),
                       pltpu.VMEM((NCHS, CHS), jnp.int32),
                       pltpu.VMEM((CHS, H), jnp.float32),
                       pltpu.VMEM((CHS, H), jnp.float32),
                       pltpu.SemaphoreType.DMA,
                       pltpu.SemaphoreType.DMA,
                       pltpu.SemaphoreType.DMA,
                       pltpu.SemaphoreType.DMA],
    )
    return gather, scatter


def _sc_gather(xt, row2, col2):
    return _sc_kernels()[0](xt, row2, col2)


def _sc_scatter(msgs, row3, zeros):
    return _sc_kernels()[1](msgs, row3, zeros)


def kernel(x, edge_index, edge_attr, batch, params):
    xpad = jnp.pad(_f32(x), ((0, NPAD - N), (0, 0)))
    bpad = jnp.pad(batch, (0, NPAD - N), constant_values=NG)
    brow = bpad.reshape(1, NPAD)
    row = edge_index[0]
    col = edge_index[1]
    row2 = row.reshape(NW, PERW)
    col2 = col.reshape(NW, PERW)
    row3 = row.reshape(NW, NCHS, CHS)
    ea = _f32(edge_attr)
    zeros_pad = jnp.zeros((NPAD, H), jnp.float32)

    xp, cnt = _encoder(xpad, brow, params)
    vn = params['vn']
    x_prev = None
    for li, lp in enumerate(params['layers']):
        vnu, vrow = _vn_stage(xp, brow, cnt, vn)
        xt, cu, sn = _node_stage(xp, brow, vnu, vrow, vn, lp)
        ni, nj = _sc_gather(xt, row2, col2)
        msgs, se = _edge_stage(ni, nj, ea, lp)
        aggp = _sc_scatter(msgs, row3, zeros_pad)
        xp = _combine_stage(xt, aggp, cu, se, sn, lp, x_prev)
        x_prev = xp

    addp, maxp = _pool_stage(xp, brow)
    logits, conf, emb = _heads_stage(addp, maxp, cnt, params['clf'],
                                     params['nf'])
    return logits, conf, emb


# speedup vs baseline: 2.8965x; 1.0006x over previous
import functools
import math

import jax
import jax.numpy as jnp
from jax import lax
from jax.experimental import pallas as pl
from jax.experimental.pallas import tpu as pltpu
from jax.experimental.pallas import tpu_sc as plsc

H = 128
NG = 64
N = 10000
NE = 320000
NB = 128
NPAD = 10240
NBLK = NPAD // NB
EB = 1280
EBLK = NE // EB
NW = 32
PERW = NE // NW
CH = 80
NCH = PERW // CH
ROWS_PER_TILE = NPAD // 16

_BNS = 1.0 / math.sqrt(1.0 + 1e-5)


def _f32(x):
    return x.astype(jnp.float32)


def _dot(a, b):
    return jax.lax.dot_general(a, b, (((1,), (0,)), ((), ())),
                               precision=jax.lax.Precision.HIGHEST,
                               preferred_element_type=jnp.float32)


def _dotd(a, b):
    return jax.lax.dot_general(a, b, (((1,), (0,)), ((), ())),
                               preferred_element_type=jnp.float32)


def _enc_body(xb, brow, iew, ieb, ieg, iebe, x0, cnt, acc):
    i = pl.program_id(0)

    @pl.when(i == 0)
    def _():
        acc[...] = jnp.zeros_like(acc)

    oh = (lax.broadcasted_iota(jnp.int32, (NG, NB), 0) == brow[...]
          ).astype(jnp.float32)
    acc[...] += jnp.sum(oh, axis=1, keepdims=True)
    y = _dotd(xb[...], iew[...]) + ieb[...]
    x0[...] = jnp.maximum(y * (ieg[...] * _BNS) + iebe[...], 0.0)

    @pl.when(i == NBLK - 1)
    def _():
        cnt[...] = jnp.maximum(acc[...], 1.0)


def _encoder(xp, brow, p):
    return pl.pallas_call(
        _enc_body,
        grid=(NBLK,),
        in_specs=[
            pl.BlockSpec((NB, H), lambda i: (i, 0)),
            pl.BlockSpec((1, NB), lambda i: (0, i)),
            pl.BlockSpec((H, H), lambda i: (0, 0)),
            pl.BlockSpec((1, H), lambda i: (0, 0)),
            pl.BlockSpec((1, H), lambda i: (0, 0)),
            pl.BlockSpec((1, H), lambda i: (0, 0)),
        ],
        out_specs=[
            pl.BlockSpec((NB, H), lambda i: (i, 0)),
            pl.BlockSpec((NG, 1), lambda i: (0, 0)),
        ],
        out_shape=[
            jax.ShapeDtypeStruct((NPAD, H), jnp.float32),
            jax.ShapeDtypeStruct((NG, 1), jnp.float32),
        ],
        scratch_shapes=[pltpu.VMEM((NG, 1), jnp.float32)],
    )(xp, brow, p['ie_w'], p['ie_b'].reshape(1, H),
      p['ie_g'].reshape(1, H), p['ie_be'].reshape(1, H))


def _vn_body(xb, brow, cnt, vemb, w1, b1, g, be, w2, b2, a1wb, a1b,
             vnu_out, vrow_out, acc):
    i = pl.program_id(0)

    @pl.when(i == 0)
    def _():
        acc[...] = jnp.zeros_like(acc)

    oh = (lax.broadcasted_iota(jnp.int32, (NG, NB), 0) == brow[...]
          ).astype(jnp.float32)
    acc[...] += _dot(oh, xb[...])

    @pl.when(i == NBLK - 1)
    def _():
        vn_in = acc[...] / cnt[...]
        z = vemb[...] + vn_in
        h = jnp.maximum((_dotd(z, w1[...]) + b1[...]) * (g[...] * _BNS)
                        + be[...], 0.0)
        vnu = _dotd(h, w2[...]) + b2[...]
        vnu_out[...] = vnu
        vrow_out[...] = _dotd(vnu, a1wb[...]) + a1b[...]


def _vn_stage(xp, brow, cnt, vn):
    m = vn['mlp']
    return pl.pallas_call(
        _vn_body,
        grid=(NBLK,),
        in_specs=[
            pl.BlockSpec((NB, H), lambda i: (i, 0)),
            pl.BlockSpec((1, NB), lambda i: (0, i)),
            pl.BlockSpec((NG, 1), lambda i: (0, 0)),
            pl.BlockSpec((1, H), lambda i: (0, 0)),
            pl.BlockSpec((H, 2 * H), lambda i: (0, 0)),
            pl.BlockSpec((1, 2 * H), lambda i: (0, 0)),
            pl.BlockSpec((1, 2 * H), lambda i: (0, 0)),
            pl.BlockSpec((1, 2 * H), lambda i: (0, 0)),
            pl.BlockSpec((2 * H, H), lambda i: (0, 0)),
            pl.BlockSpec((1, H), lambda i: (0, 0)),
            pl.BlockSpec((H, H), lambda i: (0, 0)),
            pl.BlockSpec((1, H), lambda i: (0, 0)),
        ],
        out_specs=[
            pl.BlockSpec((NG, H), lambda i: (0, 0)),
            pl.BlockSpec((NG, H), lambda i: (0, 0)),
        ],
        out_shape=[
            jax.ShapeDtypeStruct((NG, H), jnp.float32),
            jax.ShapeDtypeStruct((NG, H), jnp.float32),
        ],
        scratch_shapes=[pltpu.VMEM((NG, H), jnp.float32)],
    )(xp, brow, cnt, vn['emb'], m['w1'], m['b1'].reshape(1, 2 * H),
      m['g'].reshape(1, 2 * H), m['be'].reshape(1, 2 * H), m['w2'],
      m['b2'].reshape(1, H), vn['a1w'][H:, :], vn['a1b'].reshape(1, H))


def _node_body(xb, brow, vnu, vrow, a1wt, a2w, a2b, new, neb, neg, nebe,
               xt_out, cu_out, sn_out, acc):
    i = pl.program_id(0)

    @pl.when(i == 0)
    def _():
        acc[...] = jnp.zeros_like(acc)

    oh = (lax.broadcasted_iota(jnp.int32, (NG, NB), 0) == brow[...]
          ).astype(jnp.float32)
    oh2 = jnp.transpose(oh)
    valid = jnp.sum(oh2, axis=1, keepdims=True)
    vexp = _dot(oh2, vnu[...])
    t = jnp.tanh(_dotd(xb[...], a1wt[...]) + _dot(oh2, vrow[...]))
    e = jnp.exp(_dotd(t, a2w[...]) + a2b[...]) * valid
    cu_out[...] = vexp * e
    acc[...] += jnp.sum(e, axis=0, keepdims=True).sum(axis=1, keepdims=True)
    y = _dotd(xb[...], new[...]) + neb[...]
    xt_out[...] = jnp.maximum(y * (neg[...] * _BNS) + nebe[...], 0.0)

    @pl.when(i == NBLK - 1)
    def _():
        sn_out[...] = acc[...]


def _node_stage(xp, brow, vnu, vrow, vn, lp):
    return pl.pallas_call(
        _node_body,
        grid=(NBLK,),
        in_specs=[
            pl.BlockSpec((NB, H), lambda i: (i, 0)),
            pl.BlockSpec((1, NB), lambda i: (0, i)),
            pl.BlockSpec((NG, H), lambda i: (0, 0)),
            pl.BlockSpec((NG, H), lambda i: (0, 0)),
            pl.BlockSpec((H, H), lambda i: (0, 0)),
            pl.BlockSpec((H, 1), lambda i: (0, 0)),
            pl.BlockSpec((1, 1), lambda i: (0, 0)),
            pl.BlockSpec((H, H), lambda i: (0, 0)),
            pl.BlockSpec((1, H), lambda i: (0, 0)),
            pl.BlockSpec((1, H), lambda i: (0, 0)),
            pl.BlockSpec((1, H), lambda i: (0, 0)),
        ],
        out_specs=[
            pl.BlockSpec((NB, H), lambda i: (i, 0)),
            pl.BlockSpec((NB, H), lambda i: (i, 0)),
            pl.BlockSpec((1, 1), lambda i: (0, 0)),
        ],
        out_shape=[
            jax.ShapeDtypeStruct((NPAD, H), jnp.float32),
            jax.ShapeDtypeStruct((NPAD, H), jnp.float32),
            jax.ShapeDtypeStruct((1, 1), jnp.float32),
        ],
        scratch_shapes=[pltpu.VMEM((1, 1), jnp.float32)],
    )(xp, brow, vnu, vrow, vn['a1w'][:H, :], vn['a2w'],
      vn['a2b'].reshape(1, 1), lp['ne_w'], lp['ne_b'].reshape(1, H),
      lp['ne_g'].reshape(1, H), lp['ne_be'].reshape(1, H))


def _edge_body(nib, njb, eab, eew, eeb, eeg, eebe, a1wt, a1wb, a1b, a2w, a2b,
               msg_out, se_out, acc):
    i = pl.program_id(0)

    @pl.when(i == 0)
    def _():
        acc[...] = jnp.zeros_like(acc)

    ef = jnp.maximum((_dotd(eab[...], eew[...]) + eeb[...])
                     * (eeg[...] * _BNS) + eebe[...], 0.0)
    ni_ef = nib[...] + ef
    t = jnp.tanh(_dotd(ni_ef, a1wt[...]) + _dotd(njb[...], a1wb[...])
                 + a1b[...])
    lg = _dotd(t, a2w[...]) + a2b[...]
    e = jnp.exp(jnp.where(lg > 0, lg, 0.2 * lg))
    msg_out[...] = (njb[...] + ef) * e
    acc[...] += jnp.sum(e, axis=0, keepdims=True).sum(axis=1, keepdims=True)

    @pl.when(i == EBLK - 1)
    def _():
        se_out[...] = acc[...]


def _edge_stage(ni, nj, ea, lp):
    return pl.pallas_call(
        _edge_body,
        grid=(EBLK,),
        in_specs=[
            pl.BlockSpec((EB, H), lambda i: (i, 0)),
            pl.BlockSpec((EB, H), lambda i: (i, 0)),
            pl.BlockSpec((EB, 16), lambda i: (i, 0)),
            pl.BlockSpec((16, H), lambda i: (0, 0)),
            pl.BlockSpec((1, H), lambda i: (0, 0)),
            pl.BlockSpec((1, H), lambda i: (0, 0)),
            pl.BlockSpec((1, H), lambda i: (0, 0)),
            pl.BlockSpec((H, H), lambda i: (0, 0)),
            pl.BlockSpec((H, H), lambda i: (0, 0)),
            pl.BlockSpec((1, H), lambda i: (0, 0)),
            pl.BlockSpec((H, 1), lambda i: (0, 0)),
            pl.BlockSpec((1, 1), lambda i: (0, 0)),
        ],
        out_specs=[
            pl.BlockSpec((EB, H), lambda i: (i, 0)),
            pl.BlockSpec((1, 1), lambda i: (0, 0)),
        ],
        out_shape=[
            jax.ShapeDtypeStruct((NE, H), jnp.float32),
            jax.ShapeDtypeStruct((1, 1), jnp.float32),
        ],
        scratch_shapes=[pltpu.VMEM((1, 1), jnp.float32)],
    )(ni, nj, ea, lp['ee_w'], lp['ee_b'].reshape(1, H),
      lp['ee_g'].reshape(1, H), lp['ee_be'].reshape(1, H),
      lp['a1w'][:H, :], lp['a1w'][H:, :], lp['a1b'].reshape(1, H),
      lp['a2w'], lp['a2b'].reshape(1, 1))


def _combine_impl(xtb, a0b, a1b_, cub, se, sn, eps, w1, b1, g, be, w2, b2,
                  xpb, out):
    h0 = ((1.0 + eps[...]) * xtb[...]
          + (a0b[...] + a1b_[...]) * (1.0 / se[...])
          + cub[...] * (1.0 / sn[...]))
    h = jnp.maximum((_dotd(h0, w1[...]) + b1[...]) * (g[...] * _BNS)
                    + be[...], 0.0)
    o = _dotd(h, w2[...]) + b2[...]
    if xpb is not None:
        o = o + xpb[...]
    out[...] = o


def _combine_body_first(xtb, a0b, a1b_, cub, se, sn, eps, w1, b1, g, be,
                        w2, b2, out):
    _combine_impl(xtb, a0b, a1b_, cub, se, sn, eps, w1, b1, g, be, w2, b2,
                  None, out)


def _combine_stage(xt, aggp, cu, se, sn, lp, x_prev):
    m = lp['mlp']
    has_prev = x_prev is not None
    body = _combine_impl if has_prev else _combine_body_first
    in_specs = [
        pl.BlockSpec((NB, H), lambda i: (i, 0)),
        pl.BlockSpec((NB, H), lambda i: (i, 0)),
        pl.BlockSpec((NB, H), lambda i: (i + NBLK, 0)),
        pl.BlockSpec((NB, H), lambda i: (i, 0)),
        pl.BlockSpec((1, 1), lambda i: (0, 0)),
        pl.BlockSpec((1, 1), lambda i: (0, 0)),
        pl.BlockSpec((1, 1), lambda i: (0, 0)),
        pl.BlockSpec((H, 2 * H), lambda i: (0, 0)),
        pl.BlockSpec((1, 2 * H), lambda i: (0, 0)),
        pl.BlockSpec((1, 2 * H), lambda i: (0, 0)),
        pl.BlockSpec((1, 2 * H), lambda i: (0, 0)),
        pl.BlockSpec((2 * H, H), lambda i: (0, 0)),
        pl.BlockSpec((1, H), lambda i: (0, 0)),
    ]
    args = [xt, aggp, aggp, cu, se, sn, lp['eps'].reshape(1, 1),
            m['w1'], m['b1'].reshape(1, 2 * H), m['g'].reshape(1, 2 * H),
            m['be'].reshape(1, 2 * H), m['w2'], m['b2'].reshape(1, H)]
    if has_prev:
        in_specs.append(pl.BlockSpec((NB, H), lambda i: (i, 0)))
        args.append(x_prev)
    return pl.pallas_call(
        body,
        grid=(NBLK,),
        in_specs=in_specs,
        out_specs=pl.BlockSpec((NB, H), lambda i: (i, 0)),
        out_shape=jax.ShapeDtypeStruct((NPAD, H), jnp.float32),
    )(*args)


def _pool_body(xb, brow, addo, maxo, accs, accm):
    i = pl.program_id(0)

    @pl.when(i == 0)
    def _():
        accs[...] = jnp.zeros_like(accs)
        accm[...] = jnp.full_like(accm, -3e38)

    oh = (lax.broadcasted_iota(jnp.int32, (NG, NB), 0) == brow[...]
          ).astype(jnp.float32)
    accs[...] += _dot(oh, xb[...])
    masked = jnp.where(oh[:, :, None] > 0.5, xb[...][None, :, :], -3e38)
    accm[...] = jnp.maximum(accm[...], jnp.max(masked, axis=1))

    @pl.when(i == NBLK - 1)
    def _():
        addo[...] = accs[...]
        maxo[...] = accm[...]


def _pool_stage(xp, brow):
    return pl.pallas_call(
        _pool_body,
        grid=(NBLK,),
        in_specs=[
            pl.BlockSpec((NB, H), lambda i: (i, 0)),
            pl.BlockSpec((1, NB), lambda i: (0, i)),
        ],
        out_specs=[
            pl.BlockSpec((NG, H), lambda i: (0, 0)),
            pl.BlockSpec((NG, H), lambda i: (0, 0)),
        ],
        out_shape=[
            jax.ShapeDtypeStruct((NG, H), jnp.float32),
            jax.ShapeDtypeStruct((NG, H), jnp.float32),
        ],
        scratch_shapes=[pltpu.VMEM((NG, H), jnp.float32),
                        pltpu.VMEM((NG, H), jnp.float32)],
    )(xp, brow)


def _heads_body(addp, maxp, cnt, cw1, cb1, cg1, cbe1, cw2, cb2, cg2, cbe2,
                cw3, cb3, ncw1, ncb1, ncg, ncbe, ncw2, ncb2, nnw1, nnb1,
                nng, nnbe, nnw2, nnb2, fw1, fb1, fw2, fb2,
                logits, conf, emb):
    add = addp[...]
    meanp = add / cnt[...]
    emb_v = jnp.concatenate([add, meanp, maxp[...]], axis=1)
    emb[...] = emb_v
    h = jnp.maximum((_dotd(emb_v, cw1[...]) + cb1[...]) * (cg1[...] * _BNS)
                    + cbe1[...], 0.0)
    h = jnp.maximum((_dotd(h, cw2[...]) + cb2[...]) * (cg2[...] * _BNS)
                    + cbe2[...], 0.0)
    logits[...] = _dotd(h, cw3[...]) + cb3[...]
    cl = _dotd(jnp.maximum((_dotd(emb_v, ncw1[...]) + ncb1[...])
                          * (ncg[...] * _BNS) + ncbe[...], 0.0),
              ncw2[...]) + ncb2[...]
    no = _dotd(jnp.maximum((_dotd(emb_v, nnw1[...]) + nnb1[...])
                          * (nng[...] * _BNS) + nnbe[...], 0.0),
              nnw2[...]) + nnb2[...]
    comb = jnp.concatenate([cl, no], axis=1)
    f = jnp.maximum(_dotd(comb, fw1[...]) + fb1[...], 0.0)
    z = _dotd(f, fw2[...]) + fb2[...]
    conf[...] = 1.0 / (1.0 + jnp.exp(-z))


def _heads_stage(addp, maxp, cnt, clf, nf):
    PD = 3 * H
    h2, h4 = PD // 2, PD // 4

    def fullspec(shape):
        return pl.BlockSpec(shape, lambda: tuple(0 for _ in shape))

    args = [addp, maxp, cnt,
            clf['w1'], clf['b1'].reshape(1, H), clf['g1'].reshape(1, H),
            clf['be1'].reshape(1, H), clf['w2'], clf['b2'].reshape(1, H // 2),
            clf['g2'].reshape(1, H // 2), clf['be2'].reshape(1, H // 2),
            clf['w3'], clf['b3'].reshape(1, 6),
            nf['cw1'], nf['cb1'].reshape(1, h2), nf['cg'].reshape(1, h2),
            nf['cbe'].reshape(1, h2), nf['cw2'], nf['cb2'].reshape(1, h4),
            nf['nw1'], nf['nb1'].reshape(1, h2), nf['ng'].reshape(1, h2),
            nf['nbe'].reshape(1, h2), nf['nw2'], nf['nb2'].reshape(1, h4),
            nf['fw1'], nf['fb1'].reshape(1, h4), nf['fw2'],
            nf['fb2'].reshape(1, 1)]
    return pl.pallas_call(
        _heads_body,
        in_specs=[fullspec(a.shape) for a in args],
        out_specs=[fullspec((NG, 6)), fullspec((NG, 1)), fullspec((NG, PD))],
        out_shape=[
            jax.ShapeDtypeStruct((NG, 6), jnp.float32),
            jax.ShapeDtypeStruct((NG, 1), jnp.float32),
            jax.ShapeDtypeStruct((NG, PD), jnp.float32),
        ],
    )(*args)


CHG = 128
NCHG = PERW // CHG
GTAIL = PERW - NCHG * CHG
CHS = 80
NCHS = PERW // CHS


def _sc_gather_body(xt_hbm, row_hbm, col_hbm, ni_hbm, nj_hbm,
                    idx_r, idx_c, g0, g1, g2, g3,
                    sg0, sg1, sg2, sg3, ss0, ss1, ss2, ss3):
    wid = lax.axis_index("s") * 2 + lax.axis_index("c")
    base_w = wid * PERW
    pltpu.sync_copy(row_hbm.at[wid], idx_r)
    pltpu.sync_copy(col_hbm.at[wid], idx_c)
    phases = ((idx_r, ni_hbm, (g0, g1), (sg0, sg1), (ss0, ss1)),
              (idx_c, nj_hbm, (g2, g3), (sg2, sg3), (ss2, ss3)))
    for idx, out, bufs, gsems, osems in phases:
        pltpu.async_copy(xt_hbm.at[idx.at[pl.ds(0, CHG)]], bufs[0], gsems[0])
        pltpu.async_copy(xt_hbm.at[idx.at[pl.ds(CHG, CHG)]], bufs[1],
                         gsems[1])

    def step(i, carry):
        k0 = i * 2
        for idx, out, bufs, gsems, osems in phases:
            for b in range(2):
                k = k0 + b
                buf, gs, os = bufs[b], gsems[b], osems[b]
                pltpu.make_async_copy(xt_hbm.at[idx.at[pl.ds(0, CHG)]],
                                      buf, gs).wait()
                pltpu.async_copy(buf, out.at[pl.ds(base_w + k * CHG, CHG)],
                                 os)

                @pl.when(k + 2 < NCHG)
                def _():
                    pltpu.make_async_copy(
                        buf, out.at[pl.ds(base_w, CHG)], os).wait()
                    pltpu.async_copy(
                        xt_hbm.at[idx.at[pl.ds((k + 2) * CHG, CHG)]], buf, gs)
        return carry

    lax.fori_loop(0, NCHG // 2, step, 0)
    for idx, out, bufs, gsems, osems in phases:
        for b in range(2):
            pltpu.make_async_copy(bufs[b], out.at[pl.ds(base_w, CHG)],
                                  osems[b]).wait()
        tail = bufs[0].at[pl.ds(0, GTAIL)]
        pltpu.async_copy(xt_hbm.at[idx.at[pl.ds(NCHG * CHG, GTAIL)]],
                         tail, gsems[0]).wait()
        pltpu.sync_copy(tail, out.at[pl.ds(base_w + NCHG * CHG, GTAIL)])


def _sc_scatter_body(msgs_hbm, row3_hbm, zeros_hbm, out_hbm,
                     shared, idx3, m0, m1, sl0, sl1, sa0, sa1):
    c = lax.axis_index("c")
    s = lax.axis_index("s")
    wid = s * 2 + c
    base_w = wid * PERW
    pltpu.sync_copy(row3_hbm.at[wid], idx3)
    pltpu.sync_copy(zeros_hbm.at[pl.ds(s * ROWS_PER_TILE, ROWS_PER_TILE)],
                    shared.at[pl.ds(s * ROWS_PER_TILE, ROWS_PER_TILE)])
    plsc.subcore_barrier()
    bufs = (m0, m1)
    lsems = (sl0, sl1)
    asems = (sa0, sa1)
    pltpu.async_copy(msgs_hbm.at[pl.ds(base_w, CHS)], m0, sl0)
    pltpu.async_copy(msgs_hbm.at[pl.ds(base_w + CHS, CHS)], m1, sl1)

    def step(i, carry):
        k0 = i * 2
        for b in range(2):
            k = k0 + b
            buf, sl, sa = bufs[b], lsems[b], asems[b]
            pltpu.make_async_copy(msgs_hbm.at[pl.ds(base_w, CHS)],
                                  buf, sl).wait()
            pltpu.async_copy(buf, shared.at[idx3.at[k]], sa, add=True)

            @pl.when(k + 2 < NCHS)
            def _():
                pltpu.make_async_copy(buf, shared.at[idx3.at[k]], sa).wait()
                pltpu.async_copy(
                    msgs_hbm.at[pl.ds(base_w + (k + 2) * CHS, CHS)], buf, sl)
        return carry

    lax.fori_loop(0, NCHS // 2, step, 0)
    pltpu.make_async_copy(msgs_hbm.at[pl.ds(base_w, CHS)], m0, sl0).wait()
    pltpu.async_copy(m0, shared.at[idx3.at[NCHS - 1]], sa0, add=True)
    pltpu.make_async_copy(m0, shared.at[idx3.at[0]], sa0).wait()
    pltpu.make_async_copy(m1, shared.at[idx3.at[0]], sa1).wait()
    plsc.subcore_barrier()
    pltpu.sync_copy(
        shared.at[pl.ds(s * ROWS_PER_TILE, ROWS_PER_TILE)],
        out_hbm.at[pl.ds(c * NPAD + s * ROWS_PER_TILE, ROWS_PER_TILE)])


@functools.cache
def _sc_kernels():
    mesh = plsc.VectorSubcoreMesh(core_axis_name="c", subcore_axis_name="s")
    gather = pl.kernel(
        _sc_gather_body,
        out_type=(jax.ShapeDtypeStruct((NE, H), jnp.float32),
                  jax.ShapeDtypeStruct((NE, H), jnp.float32)),
        mesh=mesh,
        scratch_types=[pltpu.VMEM((PERW,), jnp.int32),
                       pltpu.VMEM((PERW,), jnp.int32)]
                      + [pltpu.VMEM((CHG, H), jnp.float32)] * 4
                      + [pltpu.SemaphoreType.DMA] * 8,
    )
    scatter = pl.kernel(
        _sc_scatter_body,
        out_type=jax.ShapeDtypeStruct((2 * NPAD, H), jnp.float32),
        mesh=mesh,
        scratch_types=[pltpu.VMEM_SHARED((NPAD, H), jnp.float32),
                       pltpu.VMEM((NCHS, CHS), jnp.int32)]
                      + [pltpu.VMEM((CHS, H), jnp.float32)] * 2
                      + [pltpu.SemaphoreType.DMA] * 4,
    )
    return gather, scatter


def _sc_gather(xt, row2, col2):
    return _sc_kernels()[0](xt, row2, col2)


def _sc_scatter(msgs, row3, zeros):
    return _sc_kernels()[1](msgs, row3, zeros)


def kernel(x, edge_index, edge_attr, batch, params):
    xpad = jnp.pad(_f32(x), ((0, NPAD - N), (0, 0)))
    bpad = jnp.pad(batch, (0, NPAD - N), constant_values=NG)
    brow = bpad.reshape(1, NPAD)
    row = edge_index[0]
    col = edge_index[1]
    row2 = row.reshape(NW, PERW)
    col2 = col.reshape(NW, PERW)
    row3 = row.reshape(NW, NCHS, CHS)
    ea = _f32(edge_attr)
    zeros_pad = jnp.zeros((NPAD, H), jnp.float32)

    xp, cnt = _encoder(xpad, brow, params)
    vn = params['vn']
    x_prev = None
    for li, lp in enumerate(params['layers']):
        vnu, vrow = _vn_stage(xp, brow, cnt, vn)
        xt, cu, sn = _node_stage(xp, brow, vnu, vrow, vn, lp)
        ni, nj = _sc_gather(xt, row2, col2)
        msgs, se = _edge_stage(ni, nj, ea, lp)
        aggp = _sc_scatter(msgs, row3, zeros_pad)
        xp = _combine_stage(xt, aggp, cu, se, sn, lp, x_prev)
        x_prev = xp

    addp, maxp = _pool_stage(xp, brow)
    logits, conf, emb = _heads_stage(addp, maxp, cnt, params['clf'],
                                     params['nf'])
    return logits, conf, emb
